# baseline (speedup 1.0000x reference)
import functools

import jax
import jax.numpy as jnp
from jax import lax
from jax.experimental import pallas as pl
from jax.experimental.pallas import tpu as pltpu
from jax.experimental.pallas import tpu_sc as plsc

_EMBED = 64
_BATCH = 16384
_NW = 32
_CH = 128
_ROWS_PER_W = _BATCH // _NW
_NCH = _ROWS_PER_W // _CH
_GROUPS = _CH // 16


def _make_sc_kernel():
    mesh = plsc.VectorSubcoreMesh(core_axis_name="c", subcore_axis_name="s")

    @functools.partial(
        pl.kernel,
        mesh=mesh,
        compiler_params=pltpu.CompilerParams(
            needs_layout_passes=False, use_tc_tiling_on_sc=False),
        out_type=jax.ShapeDtypeStruct((_BATCH,), jnp.float32),
        scratch_types=[
            pltpu.VMEM((_NCH, _CH), jnp.int32),
            pltpu.VMEM((_NCH, _CH), jnp.int32),
            pltpu.VMEM((_CH, _EMBED), jnp.float32),
            pltpu.VMEM((_CH, _EMBED), jnp.float32),
            pltpu.VMEM((_CH, _EMBED), jnp.float32),
            pltpu.VMEM((_CH, _EMBED), jnp.float32),
            pltpu.VMEM((2, 16), jnp.float32),
            pltpu.VMEM((_ROWS_PER_W,), jnp.float32),
            pltpu.SemaphoreType.DMA,
            pltpu.SemaphoreType.DMA,
        ],
    )
    def skipgram(widx_hbm, cidx_hbm, table_hbm, scale_hbm, out_hbm,
                 widx_v, cidx_v, w0, w1, c0, c1, scale_v, outbuf,
                 sem0, sem1):
        wid = lax.axis_index("s") * 2 + lax.axis_index("c")

        pltpu.sync_copy(widx_hbm.at[pl.ds(wid * _NCH, _NCH)], widx_v)
        pltpu.sync_copy(cidx_hbm.at[pl.ds(wid * _NCH, _NCH)], cidx_v)
        pltpu.sync_copy(scale_hbm, scale_v)

        wbufs = [w0, w1]
        cbufs = [c0, c1]
        sems = [sem0, sem1]

        def fire(k):
            slot = k % 2
            cw = pltpu.async_copy(table_hbm.at[widx_v.at[k]], wbufs[slot],
                                  sems[slot])
            cc = pltpu.async_copy(table_hbm.at[cidx_v.at[k]], cbufs[slot],
                                  sems[slot])
            return cw, cc

        inflight = fire(0)

        wv = scale_v[0, :]
        bv = scale_v[1, :]
        iota16 = lax.iota(jnp.int32, 16)

        for k in range(_NCH):
            slot = k % 2
            cw, cc = inflight
            cw.wait()
            cc.wait()
            if k + 1 < _NCH:
                inflight = fire(k + 1)

            wrows = wbufs[slot]
            crows = cbufs[slot]

            def group_body(g, _, wrows=wrows, crows=crows, base=k * _CH):
                rows = g * 16 + iota16
                accs = [jnp.zeros((16,), jnp.float32) for _ in range(4)]
                for e in range(_EMBED):
                    col = jnp.full((16,), e, jnp.int32)
                    a = plsc.load_gather(wrows, [rows, col])
                    b = plsc.load_gather(crows, [rows, col])
                    accs[e % 4] = accs[e % 4] + a * b
                acc = (accs[0] + accs[1]) + (accs[2] + accs[3])
                z = acc * wv + bv
                s = 1.0 / (1.0 + jnp.exp(-z))
                outbuf[pl.ds(base + g * 16, 16)] = s
                return 0

            lax.fori_loop(0, _GROUPS, group_body, 0)

        pltpu.sync_copy(outbuf,
                        out_hbm.at[pl.ds(wid * _ROWS_PER_W, _ROWS_PER_W)])

    return skipgram


_sc_skipgram = _make_sc_kernel()


@jax.jit
def kernel(word, context, table, dense_w, dense_b):
    widx = word.reshape(_NW * _NCH, _CH).astype(jnp.int32)
    cidx = context.reshape(_NW * _NCH, _CH).astype(jnp.int32)
    scale = jnp.stack([
        jnp.broadcast_to(dense_w.reshape(()), (16,)),
        jnp.broadcast_to(dense_b.reshape(()), (16,)),
    ]).astype(jnp.float32)
    out = _sc_skipgram(widx, cidx, table, scale)
    return out.reshape(_BATCH, 1)


# speedup vs baseline: 1.0018x; 1.0018x over previous
import functools

import jax
import jax.numpy as jnp
from jax import lax
from jax.experimental import pallas as pl
from jax.experimental.pallas import tpu as pltpu
from jax.experimental.pallas import tpu_sc as plsc

_EMBED = 64
_BATCH = 16384
_NW = 32
_CH = 128
_ROWS_PER_W = _BATCH // _NW
_NCH = _ROWS_PER_W // _CH
_GROUPS = _CH // 16


def _make_sc_kernel():
    mesh = plsc.VectorSubcoreMesh(core_axis_name="c", subcore_axis_name="s")

    @functools.partial(
        pl.kernel,
        mesh=mesh,
        compiler_params=pltpu.CompilerParams(
            needs_layout_passes=False, use_tc_tiling_on_sc=True),
        out_type=jax.ShapeDtypeStruct((_BATCH,), jnp.float32),
        scratch_types=[
            pltpu.VMEM((_ROWS_PER_W,), jnp.int32),
            pltpu.VMEM((_ROWS_PER_W,), jnp.int32),
            pltpu.VMEM((_CH,), jnp.int32),
            pltpu.VMEM((_CH,), jnp.int32),
            pltpu.VMEM((_CH,), jnp.int32),
            pltpu.VMEM((_CH,), jnp.int32),
            pltpu.VMEM((_CH,), jnp.int32),
            pltpu.VMEM((_CH,), jnp.int32),
            pltpu.VMEM((_CH,), jnp.int32),
            pltpu.VMEM((_CH,), jnp.int32),
            pltpu.VMEM((_CH, 128), jnp.float32),
            pltpu.VMEM((_CH, 128), jnp.float32),
            pltpu.VMEM((_CH, 128), jnp.float32),
            pltpu.VMEM((_CH, 128), jnp.float32),
            pltpu.VMEM((8, 128), jnp.float32),
            pltpu.VMEM((_ROWS_PER_W,), jnp.float32),
            pltpu.SemaphoreType.DMA,
            pltpu.SemaphoreType.DMA,
        ],
    )
    def skipgram(widx_hbm, cidx_hbm, tablep_hbm, scale_hbm, out_hbm,
                 widx_v, cidx_v, wp0, wp1, wp2, wp3, cp0, cp1, cp2, cp3,
                 w0, w1, c0, c1, scale_v, outbuf, sem0, sem1):
        wid = lax.axis_index("s") * 2 + lax.axis_index("c")
        base = pl.multiple_of(wid * _ROWS_PER_W, _ROWS_PER_W)

        pltpu.sync_copy(widx_hbm.at[pl.ds(base, _ROWS_PER_W)], widx_v)
        pltpu.sync_copy(cidx_hbm.at[pl.ds(base, _ROWS_PER_W)], cidx_v)
        pltpu.sync_copy(scale_hbm, scale_v)

        wpids = [wp0, wp1, wp2, wp3]
        cpids = [cp0, cp1, cp2, cp3]
        for k in range(_NCH):
            for j in range(_CH // 16):
                sl = pl.ds(j * 16, 16)
                wpids[k][sl] = widx_v[pl.ds(k * _CH + j * 16, 16)] >> 1
                cpids[k][sl] = cidx_v[pl.ds(k * _CH + j * 16, 16)] >> 1

        wbufs = [w0, w1]
        cbufs = [c0, c1]
        sems = [sem0, sem1]

        def fire(k):
            slot = k % 2
            cw = pltpu.async_copy(tablep_hbm.at[wpids[k]], wbufs[slot],
                                  sems[slot])
            cc = pltpu.async_copy(tablep_hbm.at[cpids[k]], cbufs[slot],
                                  sems[slot])
            return cw, cc

        inflight = fire(0)

        wv = scale_v[0, pl.ds(0, 16)]
        bv = scale_v[1, pl.ds(0, 16)]
        iota16 = lax.iota(jnp.int32, 16)

        for k in range(_NCH):
            slot = k % 2
            cw, cc = inflight
            cw.wait()
            cc.wait()
            if k + 1 < _NCH:
                inflight = fire(k + 1)

            wrows = wbufs[slot]
            crows = cbufs[slot]

            def group_body(g, _, wrows=wrows, crows=crows, koff=k * _CH):
                rows = g * 16 + iota16
                hw = (widx_v[pl.ds(koff + g * 16, 16)] & 1) * _EMBED
                hc = (cidx_v[pl.ds(koff + g * 16, 16)] & 1) * _EMBED
                accs = [jnp.zeros((16,), jnp.float32) for _ in range(4)]
                for e in range(_EMBED):
                    a = plsc.load_gather(wrows, [rows, hw + e])
                    b = plsc.load_gather(crows, [rows, hc + e])
                    accs[e % 4] = accs[e % 4] + a * b
                acc = (accs[0] + accs[1]) + (accs[2] + accs[3])
                z = acc * wv + bv
                s = 1.0 / (1.0 + jnp.exp(-z))
                outbuf[pl.ds(koff + g * 16, 16)] = s
                return 0

            lax.fori_loop(0, _GROUPS, group_body, 0)

        pltpu.sync_copy(outbuf, out_hbm.at[pl.ds(base, _ROWS_PER_W)])

    return skipgram


_sc_skipgram = _make_sc_kernel()


@jax.jit
def kernel(word, context, table, dense_w, dense_b):
    widx = word.reshape(_BATCH).astype(jnp.int32)
    cidx = context.reshape(_BATCH).astype(jnp.int32)
    tablep = table.reshape(500000, 128)
    scale = jnp.concatenate([
        jnp.broadcast_to(dense_w.reshape(1, 1), (1, 128)),
        jnp.broadcast_to(dense_b.reshape(1, 1), (1, 128)),
        jnp.zeros((6, 128), jnp.float32),
    ]).astype(jnp.float32)
    out = _sc_skipgram(widx, cidx, tablep, scale)
    return out.reshape(_BATCH, 1)


# speedup vs baseline: 1.1713x; 1.1692x over previous
import functools

import jax
import jax.numpy as jnp
from jax import lax
from jax.experimental import pallas as pl
from jax.experimental.pallas import tpu as pltpu
from jax.experimental.pallas import tpu_sc as plsc

_VOCAB = 1000000
_EMBED = 64
_BATCH = 16384
_NW = 32
_NBF = 7812
_TAIL0 = _NBF * 128
_NBW = 245
_PAIRS = (_NBW + 1) // 2
_HCAP = 16384 + 16
_SCAP = 64
_FLUSH_AT = _SCAP - 16


def _make_phase_a():
    mesh = plsc.VectorSubcoreMesh(core_axis_name="c", subcore_axis_name="s")

    @functools.partial(
        pl.kernel,
        mesh=mesh,
        compiler_params=pltpu.CompilerParams(
            needs_layout_passes=False, use_tc_tiling_on_sc=True),
        out_type=(jax.ShapeDtypeStruct((_BATCH, 128), jnp.float32),
                  jax.ShapeDtypeStruct((_BATCH, 128), jnp.float32)),
        scratch_types=[
            pltpu.VMEM((2048,), jnp.int32),
            pltpu.VMEM((_HCAP,), jnp.int32),
            pltpu.VMEM((_HCAP,), jnp.int32),
            pltpu.VMEM((_HCAP,), jnp.int32),
            pltpu.VMEM((_HCAP,), jnp.int32),
            pltpu.VMEM((_HCAP,), jnp.int32),
            pltpu.VMEM((_EMBED, 128), jnp.float32),
            pltpu.VMEM((_EMBED, 128), jnp.float32),
            pltpu.VMEM((_EMBED, 128), jnp.float32),
            pltpu.VMEM((_SCAP, 128), jnp.float32),
            pltpu.VMEM((_SCAP, 128), jnp.float32),
            pltpu.VMEM((_SCAP,), jnp.int32),
            pltpu.VMEM((_SCAP,), jnp.int32),
            pltpu.SMEM((17,), jnp.int32),
            pltpu.SMEM((17,), jnp.int32),
            pltpu.SemaphoreType.DMA,
            pltpu.SemaphoreType.DMA,
            pltpu.SemaphoreType.DMA,
        ],
    )
    def phase_a(widx_hbm, cidx_hbm, tablet_hbm, tail_hbm,
                wrows_hbm, crows_hbm,
                idxc, whits, chits, wbkt, cbkt, match, tb0, tb1, tail_v,
                wbig, cbig, wpos, cpos, wsm, csm, semd0, semd1, semf):
        wid = lax.axis_index("s") * 2 + lax.axis_index("c")
        jlo = wid * _NBW
        lo = jlo * 128
        hi = jnp.minimum(lo + _NBW * 128, _VOCAB)
        iota16 = lax.iota(jnp.int32, 16)

        pltpu.sync_copy(tail_hbm, tail_v)

        def scan(idx_hbm, hits):
            nh = jnp.int32(0)
            for c in range(_BATCH // 2048):
                pltpu.sync_copy(idx_hbm.at[pl.ds(c * 2048, 2048)], idxc)

                def vbody(v, nh, c=c):
                    r = idxc[pl.ds(v * 16, 16)]
                    m = (r >= lo) & (r < hi)
                    pos = (c * 2048 + v * 16) + iota16
                    packed = (pos << 15) | (r - lo)
                    plsc.store_compressed(hits.at[pl.ds(nh, 16)], packed, mask=m)
                    return nh + jnp.max(plsc.all_reduce_population_count(m))

                nh = lax.fori_loop(0, 128, vbody, nh)
            return nh

        nhw = scan(widx_hbm, whits)
        nhc = scan(cidx_hbm, chits)

        def bucket(hits, nh, bkt, sm):
            cur = jnp.int32(0)
            nv = (nh + 15) >> 4
            for b in range(16):
                sm[b] = cur

                def vbody(v, cur, b=b):
                    h = hits[pl.ds(v * 16, 16)]
                    valid = (v * 16 + iota16) < nh
                    m = valid & (((h & 0x7FFF) >> 11) == b)
                    plsc.store_compressed(bkt.at[pl.ds(cur, 16)], h, mask=m)
                    return cur + jnp.max(plsc.all_reduce_population_count(m))

                cur = lax.fori_loop(0, nv, vbody, cur)
            sm[16] = cur

        bucket(whits, nhw, wbkt, wsm)
        bucket(chits, nhc, cbkt, csm)

        bufs = [tb0, tb1]
        sems = [semd0, semd1]

        def fire(slot, j):
            jc = jnp.minimum(j, _NBF - 1)
            off = pl.multiple_of(jc * 128, 128)
            return pltpu.async_copy(tablet_hbm.at[:, pl.ds(off, 128)],
                                    bufs[slot], sems[slot])

        def flush(big, posr, dst_hbm):
            pltpu.async_copy(
                big, dst_hbm.at[plsc.Indices(posr, ignored_value=-1)],
                semf).wait()
            neg = jnp.full((16,), -1, jnp.int32)
            for q in range(_SCAP // 16):
                posr[pl.ds(q * 16, 16)] = neg

        def process_list(jrel, src, bkt, sm, big, posr, dst_hbm, cursor):
            b = jrel >> 4
            s = sm[b]
            t = sm[b + 1]
            v0 = s >> 4
            nv = ((t + 15) >> 4) - v0

            def mbody(vv, nm):
                v = v0 + vv
                h = bkt[pl.ds(v * 16, 16)]
                k = v * 16 + iota16
                m = (k >= s) & (k < t) & (((h & 0x7FFF) >> 7) == jrel)
                plsc.store_compressed(match.at[pl.ds(nm, 16)], h, mask=m)
                return nm + jnp.max(plsc.all_reduce_population_count(m))

            nm = lax.fori_loop(0, nv, mbody, jnp.int32(0))

            def ebody(g, cur):
                cur = lax.cond(cur > _FLUSH_AT,
                               lambda: (flush(big, posr, dst_hbm),
                                        jnp.int32(0))[1],
                               lambda: cur)
                h = match[pl.ds(g * 16, 16)]
                valid = (g * 16 + iota16) < nm
                lane = h & 127
                p = h >> 15
                posr[pl.ds(cur, 16)] = jnp.where(valid, p, -1)
                rowv = cur + iota16
                for e in range(_EMBED):
                    esp = jnp.full((16,), e, jnp.int32)
                    vals = plsc.load_gather(src, [esp, lane])
                    plsc.store_scatter(big, [rowv, esp], vals)
                return cur + 16

            return lax.fori_loop(0, (nm + 15) >> 4, ebody, cursor)

        def process_block(j, src, carry):
            jrel = j - jlo
            wcur, ccur = carry
            wcur = process_list(jrel, src, wbkt, wsm, wbig, wpos,
                                wrows_hbm, wcur)
            ccur = process_list(jrel, src, cbkt, csm, cbig, cpos,
                                crows_hbm, ccur)
            return wcur, ccur

        neg = jnp.full((16,), -1, jnp.int32)
        for q in range(_SCAP // 16):
            wpos[pl.ds(q * 16, 16)] = neg
            cpos[pl.ds(q * 16, 16)] = neg

        cp0 = fire(0, jlo)
        cp1 = fire(1, jlo + 1)

        def pair(ii, carry):
            j0 = jlo + 2 * ii
            cp = pltpu.make_async_copy(
                tablet_hbm.at[:, pl.ds(pl.multiple_of(0, 128), 128)],
                tb0, semd0)
            cp.wait()
            carry = lax.cond(j0 < _NBF,
                             lambda c: process_block(j0, tb0, c),
                             lambda c: c, carry)
            fire(0, j0 + 2)
            j1 = j0 + 1
            cp = pltpu.make_async_copy(
                tablet_hbm.at[:, pl.ds(pl.multiple_of(0, 128), 128)],
                tb1, semd1)
            cp.wait()
            carry = lax.cond(j1 < _NBF,
                             lambda c: process_block(j1, tb1, c),
                             lambda c: c, carry)
            fire(1, j1 + 2)
            return carry

        carry = lax.fori_loop(0, _PAIRS, pair,
                              (jnp.int32(0), jnp.int32(0)))

        carry = lax.cond(wid == _NW - 1,
                         lambda c: process_block(jnp.int32(_NBF), tail_v, c),
                         lambda c: c, carry)

        flush(wbig, wpos, wrows_hbm)
        flush(cbig, cpos, crows_hbm)

        dummy = tablet_hbm.at[:, pl.ds(pl.multiple_of(0, 128), 128)]
        pltpu.make_async_copy(dummy, tb0, semd0).wait()
        pltpu.make_async_copy(dummy, tb1, semd1).wait()

    return phase_a


def _make_phase_b():
    mesh = plsc.VectorSubcoreMesh(core_axis_name="c", subcore_axis_name="s")
    rows_w = _BATCH // _NW
    nch = rows_w // 128

    @functools.partial(
        pl.kernel,
        mesh=mesh,
        compiler_params=pltpu.CompilerParams(
            needs_layout_passes=False, use_tc_tiling_on_sc=True),
        out_type=jax.ShapeDtypeStruct((_BATCH,), jnp.float32),
        scratch_types=[
            pltpu.VMEM((128, 128), jnp.float32),
            pltpu.VMEM((128, 128), jnp.float32),
            pltpu.VMEM((128, 128), jnp.float32),
            pltpu.VMEM((128, 128), jnp.float32),
            pltpu.VMEM((8, 128), jnp.float32),
            pltpu.VMEM((rows_w,), jnp.float32),
            pltpu.SemaphoreType.DMA,
            pltpu.SemaphoreType.DMA,
        ],
    )
    def phase_b(wrows_hbm, crows_hbm, scale_hbm, out_hbm,
                w0, w1, c0, c1, scale_v, outbuf, sem0, sem1):
        wid = lax.axis_index("s") * 2 + lax.axis_index("c")
        base = pl.multiple_of(wid * rows_w, rows_w)
        iota16 = lax.iota(jnp.int32, 16)

        pltpu.sync_copy(scale_hbm, scale_v)

        wbufs = [w0, w1]
        cbufs = [c0, c1]
        sems = [sem0, sem1]

        def fire(k):
            slot = k % 2
            off = pl.multiple_of(base + k * 128, 128)
            cw = pltpu.async_copy(wrows_hbm.at[pl.ds(off, 128)],
                                  wbufs[slot], sems[slot])
            cc = pltpu.async_copy(crows_hbm.at[pl.ds(off, 128)],
                                  cbufs[slot], sems[slot])
            return cw, cc

        inflight = fire(0)
        wv = scale_v[0, pl.ds(0, 16)]
        bv = scale_v[1, pl.ds(0, 16)]

        for k in range(nch):
            slot = k % 2
            cw, cc = inflight
            cw.wait()
            cc.wait()
            if k + 1 < nch:
                inflight = fire(k + 1)

            wrows = wbufs[slot]
            crows = cbufs[slot]

            def group_body(g, _, wrows=wrows, crows=crows, koff=k * 128):
                rows = g * 16 + iota16
                accs = [jnp.zeros((16,), jnp.float32) for _ in range(4)]
                for e in range(_EMBED):
                    esp = jnp.full((16,), e, jnp.int32)
                    a = plsc.load_gather(wrows, [rows, esp])
                    b = plsc.load_gather(crows, [rows, esp])
                    accs[e % 4] = accs[e % 4] + a * b
                acc = (accs[0] + accs[1]) + (accs[2] + accs[3])
                z = acc * wv + bv
                s = 1.0 / (1.0 + jnp.exp(-z))
                outbuf[pl.ds(koff + g * 16, 16)] = s
                return 0

            lax.fori_loop(0, 8, group_body, 0)

        pltpu.sync_copy(outbuf, out_hbm.at[pl.ds(base, rows_w)])

    return phase_b


_phase_a = _make_phase_a()
_phase_b = _make_phase_b()


@jax.jit
def kernel(word, context, table, dense_w, dense_b):
    widx = word.reshape(_BATCH).astype(jnp.int32)
    cidx = context.reshape(_BATCH).astype(jnp.int32)
    tablet = table.T
    tail = jnp.pad(table[_TAIL0:].T.astype(jnp.float32), ((0, 0), (0, 64)))
    scale = jnp.concatenate([
        jnp.broadcast_to(dense_w.reshape(1, 1), (1, 128)),
        jnp.broadcast_to(dense_b.reshape(1, 1), (1, 128)),
        jnp.zeros((6, 128), jnp.float32),
    ]).astype(jnp.float32)
    wrows, crows = _phase_a(widx, cidx, tablet, tail)
    out = _phase_b(wrows, crows, scale)
    return out.reshape(_BATCH, 1)


# speedup vs baseline: 1.9595x; 1.6729x over previous
import functools

import jax
import jax.numpy as jnp
from jax import lax
from jax.experimental import pallas as pl
from jax.experimental.pallas import tpu as pltpu
from jax.experimental.pallas import tpu_sc as plsc

_VOCAB = 1000000
_EMBED = 64
_BATCH = 16384
_NW = 32
_NBF = 7812
_TAIL0 = _NBF * 128
_NBW = 245
_PAIRS = (_NBW + 1) // 2
_HCAP = 16384 + 16
_SCAP = 64
_FLUSH_AT = _SCAP - 16


def _make_phase_a():
    mesh = plsc.VectorSubcoreMesh(core_axis_name="c", subcore_axis_name="s")

    @functools.partial(
        pl.kernel,
        mesh=mesh,
        compiler_params=pltpu.CompilerParams(
            needs_layout_passes=False, use_tc_tiling_on_sc=True),
        out_type=(jax.ShapeDtypeStruct((_BATCH, 128), jnp.float32),
                  jax.ShapeDtypeStruct((_BATCH, 128), jnp.float32)),
        scratch_types=[
            pltpu.VMEM((2048,), jnp.int32),
            pltpu.VMEM((_HCAP,), jnp.int32),
            pltpu.VMEM((_HCAP,), jnp.int32),
            pltpu.VMEM((_HCAP,), jnp.int32),
            pltpu.VMEM((_HCAP,), jnp.int32),
            pltpu.VMEM((_HCAP,), jnp.int32),
            pltpu.VMEM((_EMBED, 128), jnp.float32),
            pltpu.VMEM((_EMBED, 128), jnp.float32),
            pltpu.VMEM((_EMBED, 128), jnp.float32),
            pltpu.VMEM((_SCAP, 128), jnp.float32),
            pltpu.VMEM((_SCAP, 128), jnp.float32),
            pltpu.VMEM((_SCAP,), jnp.int32),
            pltpu.VMEM((_SCAP,), jnp.int32),
            pltpu.VMEM((128,), jnp.int32),
            pltpu.SMEM((17,), jnp.int32),
            pltpu.SMEM((17,), jnp.int32),
            pltpu.SMEM((128,), jnp.int32),
            pltpu.SemaphoreType.DMA,
            pltpu.SemaphoreType.DMA,
            pltpu.SemaphoreType.DMA,
        ],
    )
    def phase_a(widx_hbm, cidx_hbm, tablet_hbm, tail_hbm,
                wrows_hbm, crows_hbm,
                idxc, whits, chits, wbkt, cbkt, match, tb0, tb1, tail_v,
                wbig, cbig, wpos, cpos, cntv, wsm, csm, cnsm,
                semd0, semd1, semf):
        wid = lax.axis_index("s") * 2 + lax.axis_index("c")
        jlo = wid * _NBW
        lo = jlo * 128
        hi = jnp.minimum(lo + _NBW * 128, _VOCAB)
        iota16 = lax.iota(jnp.int32, 16)
        lane0 = iota16 == 0
        evs = [iota16 + 16 * k for k in range(4)]

        pltpu.sync_copy(tail_hbm, tail_v)

        def compact(nv, maskfn, valfn, dst, cursor):
            def p1(vv, _):
                cnt = plsc.all_reduce_population_count(maskfn(vv))
                cnsm[vv] = jnp.max(cnt)
                return 0
            lax.fori_loop(0, nv, p1, 0)

            def p2(vv, cur):
                m = maskfn(vv)
                plsc.store_compressed(dst.at[pl.ds(cur, 16)], valfn(vv),
                                      mask=m)
                return cur + cnsm[vv]
            return lax.fori_loop(0, nv, p2, cursor)

        def scan(idx_hbm, hits):
            nh = jnp.int32(0)
            for c in range(_BATCH // 2048):
                pltpu.sync_copy(idx_hbm.at[pl.ds(c * 2048, 2048)], idxc)

                def maskfn(v):
                    r = idxc[pl.ds(v * 16, 16)]
                    return (r >= lo) & (r < hi)

                def valfn(v, c=c):
                    r = idxc[pl.ds(v * 16, 16)]
                    pos = (c * 2048 + v * 16) + iota16
                    return (pos << 15) | (r - lo)

                nh = compact(128, maskfn, valfn, hits, nh)
            return nh

        nhw = scan(widx_hbm, whits)
        nhc = scan(cidx_hbm, chits)

        def bucket(hits, nh, bkt, sm):
            cur = jnp.int32(0)
            nv = jnp.minimum((nh + 15) >> 4, 128)
            nv2 = (nh + 15) >> 4

            def passes(vbase, nvx, cur, sm_write):
                for b in range(16):
                    if sm_write:
                        sm[b] = cur

                    def maskfn(v, b=b, vbase=vbase):
                        v = v + vbase
                        h = hits[pl.ds(v * 16, 16)]
                        valid = (v * 16 + iota16) < nh
                        return valid & (((h & 0x7FFF) >> 11) == b)

                    def valfn(v, vbase=vbase):
                        return hits[pl.ds((v + vbase) * 16, 16)]

                    cur = compact(nvx, maskfn, valfn, bkt, cur)
                return cur

            cur = passes(0, nv, cur, True)
            sm[16] = cur
            return cur

        def bucket_slow(hits, nh, bkt, sm):
            cur = jnp.int32(0)
            nv = (nh + 15) >> 4
            for b in range(16):
                sm[b] = cur

                def vbody(v, cur, b=b):
                    h = hits[pl.ds(v * 16, 16)]
                    valid = (v * 16 + iota16) < nh
                    m = valid & (((h & 0x7FFF) >> 11) == b)
                    plsc.store_compressed(bkt.at[pl.ds(cur, 16)], h, mask=m)
                    return cur + jnp.max(plsc.all_reduce_population_count(m))

                cur = lax.fori_loop(0, nv, vbody, cur)
            sm[16] = cur
            return cur

        def bucket_any(hits, nh, bkt, sm):
            lax.cond(nh <= 2048,
                     lambda: (bucket(hits, nh, bkt, sm), None)[1],
                     lambda: (bucket_slow(hits, nh, bkt, sm), None)[1])

        bucket_any(whits, nhw, wbkt, wsm)
        bucket_any(chits, nhc, cbkt, csm)

        bufs = [tb0, tb1]
        sems = [semd0, semd1]

        def fire(slot, j):
            jc = jnp.minimum(j, _NBF - 1)
            off = pl.multiple_of(jc * 128, 128)
            return pltpu.async_copy(tablet_hbm.at[:, pl.ds(off, 128)],
                                    bufs[slot], sems[slot])

        def flush(big, posr, dst_hbm):
            pltpu.async_copy(
                big, dst_hbm.at[plsc.Indices(posr, ignored_value=-1)],
                semf).wait()
            neg = jnp.full((16,), -1, jnp.int32)
            for q in range(_SCAP // 16):
                posr[pl.ds(q * 16, 16)] = neg

        def process_list(jrel, jok, src, bkt, sm, big, posr, dst_hbm,
                         cursor):
            b = jrel >> 4
            s = sm[b]
            t = sm[b + 1]
            v0 = s >> 4
            nv = ((t + 15) >> 4) - v0

            def mbody(vv, nm):
                v = v0 + vv
                h = bkt[pl.ds(v * 16, 16)]
                k = v * 16 + iota16
                m = jok & (k >= s) & (k < t) & (((h & 0x7FFF) >> 7) == jrel)
                plsc.store_compressed(match.at[pl.ds(nm, 16)], h, mask=m)
                return nm + jnp.max(plsc.all_reduce_population_count(m))

            nm = lax.fori_loop(0, nv, mbody, jnp.int32(0))

            def ebody(g, cur):
                cur = lax.cond(cur > _FLUSH_AT,
                               lambda: (flush(big, posr, dst_hbm),
                                        jnp.int32(0))[1],
                               lambda: cur)
                h = match[pl.ds(g * 16, 16)]
                valid = (g * 16 + iota16) < nm
                posr[pl.ds(cur, 16)] = jnp.where(valid, h >> 15, -1)
                nmg = jnp.minimum(16, nm - g * 16)

                def hbody(i, _, g=g):
                    hsp = plsc.load_gather(
                        match, [jnp.full((16,), 0, jnp.int32) + (g * 16 + i)])
                    lane = hsp & 127
                    for k in range(4):
                        vals = plsc.load_gather(src, [evs[k], lane])
                        big[cur + i, pl.ds(k * 16, 16)] = vals
                    return 0

                lax.fori_loop(0, nmg, hbody, 0)
                return cur + 16

            return lax.fori_loop(0, (nm + 15) >> 4, ebody, cursor)

        def process_block(j, jok, src, carry):
            jrel = j - jlo
            wcur, ccur = carry
            wcur = process_list(jrel, jok, src, wbkt, wsm, wbig, wpos,
                                wrows_hbm, wcur)
            ccur = process_list(jrel, jok, src, cbkt, csm, cbig, cpos,
                                crows_hbm, ccur)
            return wcur, ccur

        neg = jnp.full((16,), -1, jnp.int32)
        for q in range(_SCAP // 16):
            wpos[pl.ds(q * 16, 16)] = neg
            cpos[pl.ds(q * 16, 16)] = neg

        fire(0, jlo)
        fire(1, jlo + 1)

        def pair(ii, carry):
            j0 = jlo + 2 * ii
            pltpu.make_async_copy(
                tablet_hbm.at[:, pl.ds(pl.multiple_of(0, 128), 128)],
                tb0, semd0).wait()
            carry = process_block(j0, j0 < _NBF, tb0, carry)
            fire(0, j0 + 2)
            j1 = j0 + 1
            pltpu.make_async_copy(
                tablet_hbm.at[:, pl.ds(pl.multiple_of(0, 128), 128)],
                tb1, semd1).wait()
            carry = process_block(j1, j1 < _NBF, tb1, carry)
            fire(1, j1 + 2)
            return carry

        carry = lax.fori_loop(0, _PAIRS, pair,
                              (jnp.int32(0), jnp.int32(0)))

        carry = lax.cond(wid == _NW - 1,
                         lambda c: process_block(jnp.int32(_NBF), True,
                                                 tail_v, c),
                         lambda c: c, carry)

        flush(wbig, wpos, wrows_hbm)
        flush(cbig, cpos, crows_hbm)

        dummy = tablet_hbm.at[:, pl.ds(pl.multiple_of(0, 128), 128)]
        pltpu.make_async_copy(dummy, tb0, semd0).wait()
        pltpu.make_async_copy(dummy, tb1, semd1).wait()

    return phase_a


def _make_phase_b():
    mesh = plsc.VectorSubcoreMesh(core_axis_name="c", subcore_axis_name="s")
    rows_w = _BATCH // _NW
    nch = rows_w // 128

    @functools.partial(
        pl.kernel,
        mesh=mesh,
        compiler_params=pltpu.CompilerParams(
            needs_layout_passes=False, use_tc_tiling_on_sc=True),
        out_type=jax.ShapeDtypeStruct((_BATCH,), jnp.float32),
        scratch_types=[
            pltpu.VMEM((128, 128), jnp.float32),
            pltpu.VMEM((128, 128), jnp.float32),
            pltpu.VMEM((128, 128), jnp.float32),
            pltpu.VMEM((128, 128), jnp.float32),
            pltpu.VMEM((8, 128), jnp.float32),
            pltpu.VMEM((rows_w,), jnp.float32),
            pltpu.SemaphoreType.DMA,
            pltpu.SemaphoreType.DMA,
        ],
    )
    def phase_b(wrows_hbm, crows_hbm, scale_hbm, out_hbm,
                w0, w1, c0, c1, scale_v, outbuf, sem0, sem1):
        wid = lax.axis_index("s") * 2 + lax.axis_index("c")
        base = pl.multiple_of(wid * rows_w, rows_w)
        iota16 = lax.iota(jnp.int32, 16)

        pltpu.sync_copy(scale_hbm, scale_v)

        wbufs = [w0, w1]
        cbufs = [c0, c1]
        sems = [sem0, sem1]

        def fire(k):
            slot = k % 2
            off = pl.multiple_of(base + k * 128, 128)
            cw = pltpu.async_copy(wrows_hbm.at[pl.ds(off, 128)],
                                  wbufs[slot], sems[slot])
            cc = pltpu.async_copy(crows_hbm.at[pl.ds(off, 128)],
                                  cbufs[slot], sems[slot])
            return cw, cc

        inflight = fire(0)
        wv = scale_v[0, pl.ds(0, 16)]
        bv = scale_v[1, pl.ds(0, 16)]

        for k in range(nch):
            slot = k % 2
            cw, cc = inflight
            cw.wait()
            cc.wait()
            if k + 1 < nch:
                inflight = fire(k + 1)

            wrows = wbufs[slot]
            crows = cbufs[slot]

            def group_body(g, _, wrows=wrows, crows=crows, koff=k * 128):
                rows = g * 16 + iota16
                accs = [jnp.zeros((16,), jnp.float32) for _ in range(4)]
                for e in range(_EMBED):
                    esp = jnp.full((16,), e, jnp.int32)
                    a = plsc.load_gather(wrows, [rows, esp])
                    b = plsc.load_gather(crows, [rows, esp])
                    accs[e % 4] = accs[e % 4] + a * b
                acc = (accs[0] + accs[1]) + (accs[2] + accs[3])
                z = acc * wv + bv
                s = 1.0 / (1.0 + jnp.exp(-z))
                outbuf[pl.ds(koff + g * 16, 16)] = s
                return 0

            lax.fori_loop(0, 8, group_body, 0)

        pltpu.sync_copy(outbuf, out_hbm.at[pl.ds(base, rows_w)])

    return phase_b


_phase_a = _make_phase_a()
_phase_b = _make_phase_b()


@jax.jit
def kernel(word, context, table, dense_w, dense_b):
    widx = word.reshape(_BATCH).astype(jnp.int32)
    cidx = context.reshape(_BATCH).astype(jnp.int32)
    tablet = table.T
    tail = jnp.pad(table[_TAIL0:].T.astype(jnp.float32), ((0, 0), (0, 64)))
    scale = jnp.concatenate([
        jnp.broadcast_to(dense_w.reshape(1, 1), (1, 128)),
        jnp.broadcast_to(dense_b.reshape(1, 1), (1, 128)),
        jnp.zeros((6, 128), jnp.float32),
    ]).astype(jnp.float32)
    wrows, crows = _phase_a(widx, cidx, tablet, tail)
    out = _phase_b(wrows, crows, scale)
    return out.reshape(_BATCH, 1)


# speedup vs baseline: 2.3436x; 1.1960x over previous
import functools

import jax
import jax.numpy as jnp
from jax import lax
from jax.experimental import pallas as pl
from jax.experimental.pallas import tpu as pltpu
from jax.experimental.pallas import tpu_sc as plsc

_VOCAB = 1000000
_EMBED = 64
_BATCH = 16384
_NW = 32
_BS = 256
_NBF = 3906
_TAIL0 = _NBF * _BS
_NBW = 123
_PAIRS = (_NBW + 1) // 2
_HCAP = 16384 + 16
_SCAP = 64
_FLUSH_AT = _SCAP - 16


def _make_phase_a():
    mesh = plsc.VectorSubcoreMesh(core_axis_name="c", subcore_axis_name="s")

    @functools.partial(
        pl.kernel,
        mesh=mesh,
        compiler_params=pltpu.CompilerParams(
            needs_layout_passes=False, use_tc_tiling_on_sc=True),
        out_type=(jax.ShapeDtypeStruct((_BATCH, 128), jnp.float32),
                  jax.ShapeDtypeStruct((_BATCH, 128), jnp.float32)),
        scratch_types=[
            pltpu.VMEM((2048,), jnp.int32),
            pltpu.VMEM((_HCAP,), jnp.int32),
            pltpu.VMEM((_HCAP,), jnp.int32),
            pltpu.VMEM((_HCAP,), jnp.int32),
            pltpu.VMEM((_HCAP,), jnp.int32),
            pltpu.VMEM((_EMBED, _BS), jnp.float32),
            pltpu.VMEM((_EMBED, _BS), jnp.float32),
            pltpu.VMEM((_EMBED, 128), jnp.float32),
            pltpu.VMEM((_SCAP, 128), jnp.float32),
            pltpu.VMEM((_SCAP, 128), jnp.float32),
            pltpu.VMEM((_SCAP,), jnp.int32),
            pltpu.VMEM((_SCAP,), jnp.int32),
            pltpu.SMEM((17,), jnp.int32),
            pltpu.SMEM((17,), jnp.int32),
            pltpu.SMEM((1056,), jnp.int32),
            pltpu.SemaphoreType.DMA,
            pltpu.SemaphoreType.DMA,
            pltpu.SemaphoreType.DMA,
        ],
    )
    def phase_a(widx_hbm, cidx_hbm, tablet_hbm, tail_hbm,
                wrows_hbm, crows_hbm,
                idxc, whits, chits, wbkt, cbkt, tb0, tb1, tail_v,
                wbig, cbig, wpos, cpos, wsm, csm, cnsm,
                semd0, semd1, semf):
        wid = lax.axis_index("s") * 2 + lax.axis_index("c")
        jlo = wid * _NBW
        lo = jlo * _BS
        hi = jnp.minimum(lo + _NBW * _BS, _VOCAB)
        iota16 = lax.iota(jnp.int32, 16)
        evs = [iota16 + 16 * k for k in range(4)]

        pltpu.sync_copy(tail_hbm, tail_v)

        def compact(nv, maskfn, valfn, dst, cursor):
            def p1(vv, _):
                cnt = plsc.all_reduce_population_count(maskfn(vv))
                cnsm[vv] = jnp.max(cnt)
                return 0
            lax.fori_loop(0, nv, p1, 0)

            def p2(vv, cur):
                m = maskfn(vv)
                plsc.store_compressed(dst.at[pl.ds(cur, 16)], valfn(vv),
                                      mask=m)
                return cur + cnsm[vv]
            return lax.fori_loop(0, nv, p2, cursor)

        def scan(idx_hbm, hits):
            nh = jnp.int32(0)
            for c in range(_BATCH // 2048):
                pltpu.sync_copy(idx_hbm.at[pl.ds(c * 2048, 2048)], idxc)

                def maskfn(v):
                    r = idxc[pl.ds(v * 16, 16)]
                    return (r >= lo) & (r < hi)

                def valfn(v, c=c):
                    r = idxc[pl.ds(v * 16, 16)]
                    pos = (c * 2048 + v * 16) + iota16
                    return (pos << 15) | (r - lo)

                nh = compact(128, maskfn, valfn, hits, nh)
            return nh

        nhw = scan(widx_hbm, whits)
        nhc = scan(cidx_hbm, chits)

        def bucket(hits, nh, bkt, sm):
            cur = jnp.int32(0)
            nv = (nh + 15) >> 4
            for b in range(16):
                sm[b] = cur

                def maskfn(v, b=b):
                    h = hits[pl.ds(v * 16, 16)]
                    valid = (v * 16 + iota16) < nh
                    return valid & (((h & 0x7FFF) >> 11) == b)

                def valfn(v):
                    return hits[pl.ds(v * 16, 16)]

                cur = compact(nv, maskfn, valfn, bkt, cur)
            sm[16] = cur

        bucket(whits, nhw, wbkt, wsm)
        bucket(chits, nhc, cbkt, csm)
        match = whits

        bufs = [tb0, tb1]
        sems = [semd0, semd1]

        def fire(slot, j):
            jc = jnp.minimum(j, _NBF - 1)
            off = pl.multiple_of(jc * _BS, _BS)
            return pltpu.async_copy(tablet_hbm.at[:, pl.ds(off, _BS)],
                                    bufs[slot], sems[slot])

        def flush(big, posr, dst_hbm):
            pltpu.async_copy(
                big, dst_hbm.at[plsc.Indices(posr, ignored_value=-1)],
                semf).wait()
            neg = jnp.full((16,), -1, jnp.int32)
            for q in range(_SCAP // 16):
                posr[pl.ds(q * 16, 16)] = neg

        def process_list(jrel, jok, src, bkt, sm, big, posr, dst_hbm,
                         cursor):
            b = jrel >> 3
            s = sm[b]
            t = sm[b + 1]
            v0 = s >> 4
            nv = ((t + 15) >> 4) - v0

            def mmask(vv):
                v = v0 + vv
                h = bkt[pl.ds(v * 16, 16)]
                k = v * 16 + iota16
                return jok & (k >= s) & (k < t) & (
                    ((h & 0x7FFF) >> 8) == jrel)

            def mval(vv):
                return bkt[pl.ds((v0 + vv) * 16, 16)]

            nm = compact(nv, mmask, mval, match, jnp.int32(0))

            def ebody(g, cur):
                cur = lax.cond(cur > _FLUSH_AT,
                               lambda: (flush(big, posr, dst_hbm),
                                        jnp.int32(0))[1],
                               lambda: cur)
                h = match[pl.ds(g * 16, 16)]
                valid = (g * 16 + iota16) < nm
                posr[pl.ds(cur, 16)] = jnp.where(valid, h >> 15, -1)
                nmg = jnp.minimum(16, nm - g * 16)

                def hbody(i, _, g=g):
                    hsp = plsc.load_gather(
                        match, [jnp.full((16,), 0, jnp.int32) + (g * 16 + i)])
                    lane = hsp & 255
                    for k in range(4):
                        vals = plsc.load_gather(src, [evs[k], lane])
                        big[cur + i, pl.ds(k * 16, 16)] = vals
                    return 0

                lax.fori_loop(0, nmg, hbody, 0)
                return cur + 16

            return lax.fori_loop(0, (nm + 15) >> 4, ebody, cursor)

        def process_block(j, jok, src, carry):
            jrel = j - jlo
            wcur, ccur = carry
            wcur = process_list(jrel, jok, src, wbkt, wsm, wbig, wpos,
                                wrows_hbm, wcur)
            ccur = process_list(jrel, jok, src, cbkt, csm, cbig, cpos,
                                crows_hbm, ccur)
            return wcur, ccur

        neg = jnp.full((16,), -1, jnp.int32)
        for q in range(_SCAP // 16):
            wpos[pl.ds(q * 16, 16)] = neg
            cpos[pl.ds(q * 16, 16)] = neg

        fire(0, jlo)
        fire(1, jlo + 1)

        def pair(ii, carry):
            j0 = jlo + 2 * ii
            pltpu.make_async_copy(
                tablet_hbm.at[:, pl.ds(pl.multiple_of(0, _BS), _BS)],
                tb0, semd0).wait()
            carry = process_block(j0, j0 < _NBF, tb0, carry)
            fire(0, j0 + 2)
            j1 = j0 + 1
            pltpu.make_async_copy(
                tablet_hbm.at[:, pl.ds(pl.multiple_of(0, _BS), _BS)],
                tb1, semd1).wait()
            carry = process_block(j1, j1 < _NBF, tb1, carry)
            fire(1, j1 + 2)
            return carry

        carry = lax.fori_loop(0, _PAIRS, pair,
                              (jnp.int32(0), jnp.int32(0)))

        carry = lax.cond(wid == _NW - 1,
                         lambda c: process_block(jnp.int32(_NBF), True,
                                                 tail_v, c),
                         lambda c: c, carry)

        flush(wbig, wpos, wrows_hbm)
        flush(cbig, cpos, crows_hbm)

        dummy = tablet_hbm.at[:, pl.ds(pl.multiple_of(0, _BS), _BS)]
        pltpu.make_async_copy(dummy, tb0, semd0).wait()
        pltpu.make_async_copy(dummy, tb1, semd1).wait()

    return phase_a


def _make_phase_b():
    mesh = plsc.VectorSubcoreMesh(core_axis_name="c", subcore_axis_name="s")
    rows_w = _BATCH // _NW
    nch = rows_w // 128

    @functools.partial(
        pl.kernel,
        mesh=mesh,
        compiler_params=pltpu.CompilerParams(
            needs_layout_passes=False, use_tc_tiling_on_sc=True),
        out_type=jax.ShapeDtypeStruct((_BATCH,), jnp.float32),
        scratch_types=[
            pltpu.VMEM((128, 128), jnp.float32),
            pltpu.VMEM((128, 128), jnp.float32),
            pltpu.VMEM((128, 128), jnp.float32),
            pltpu.VMEM((128, 128), jnp.float32),
            pltpu.VMEM((8, 128), jnp.float32),
            pltpu.VMEM((rows_w,), jnp.float32),
            pltpu.SemaphoreType.DMA,
            pltpu.SemaphoreType.DMA,
        ],
    )
    def phase_b(wrows_hbm, crows_hbm, scale_hbm, out_hbm,
                w0, w1, c0, c1, scale_v, outbuf, sem0, sem1):
        wid = lax.axis_index("s") * 2 + lax.axis_index("c")
        base = pl.multiple_of(wid * rows_w, rows_w)
        iota16 = lax.iota(jnp.int32, 16)

        pltpu.sync_copy(scale_hbm, scale_v)

        wbufs = [w0, w1]
        cbufs = [c0, c1]
        sems = [sem0, sem1]

        def fire(k):
            slot = k % 2
            off = pl.multiple_of(base + k * 128, 128)
            cw = pltpu.async_copy(wrows_hbm.at[pl.ds(off, 128)],
                                  wbufs[slot], sems[slot])
            cc = pltpu.async_copy(crows_hbm.at[pl.ds(off, 128)],
                                  cbufs[slot], sems[slot])
            return cw, cc

        inflight = fire(0)
        wv = scale_v[0, pl.ds(0, 16)]
        bv = scale_v[1, pl.ds(0, 16)]

        for k in range(nch):
            slot = k % 2
            cw, cc = inflight
            cw.wait()
            cc.wait()
            if k + 1 < nch:
                inflight = fire(k + 1)

            wrows = wbufs[slot]
            crows = cbufs[slot]

            def group_body(g, _, wrows=wrows, crows=crows, koff=k * 128):
                rows = g * 16 + iota16
                accs = [jnp.zeros((16,), jnp.float32) for _ in range(4)]
                for e in range(_EMBED):
                    esp = jnp.full((16,), e, jnp.int32)
                    a = plsc.load_gather(wrows, [rows, esp])
                    b = plsc.load_gather(crows, [rows, esp])
                    accs[e % 4] = accs[e % 4] + a * b
                acc = (accs[0] + accs[1]) + (accs[2] + accs[3])
                z = acc * wv + bv
                s = 1.0 / (1.0 + jnp.exp(-z))
                outbuf[pl.ds(koff + g * 16, 16)] = s
                return 0

            lax.fori_loop(0, 8, group_body, 0)

        pltpu.sync_copy(outbuf, out_hbm.at[pl.ds(base, rows_w)])

    return phase_b


_phase_a = _make_phase_a()
_phase_b = _make_phase_b()


@jax.jit
def kernel(word, context, table, dense_w, dense_b):
    widx = word.reshape(_BATCH).astype(jnp.int32)
    cidx = context.reshape(_BATCH).astype(jnp.int32)
    tablet = table.T
    tail = jnp.pad(table[_TAIL0:].T.astype(jnp.float32), ((0, 0), (0, 64)))
    scale = jnp.concatenate([
        jnp.broadcast_to(dense_w.reshape(1, 1), (1, 128)),
        jnp.broadcast_to(dense_b.reshape(1, 1), (1, 128)),
        jnp.zeros((6, 128), jnp.float32),
    ]).astype(jnp.float32)
    wrows, crows = _phase_a(widx, cidx, tablet, tail)
    out = _phase_b(wrows, crows, scale)
    return out.reshape(_BATCH, 1)


# speedup vs baseline: 2.4564x; 1.0481x over previous
import functools

import jax
import jax.numpy as jnp
from jax import lax
from jax.experimental import pallas as pl
from jax.experimental.pallas import tpu as pltpu
from jax.experimental.pallas import tpu_sc as plsc

_VOCAB = 1000000
_EMBED = 64
_BATCH = 16384
_NW = 32
_BS = 256
_NBF = 3906
_TAIL0 = _NBF * _BS
_NBW = 123
_PAIRS = (_NBW + 1) // 2
_NIDX = 2 * _BATCH
_HCAP = _NIDX + 16
_SCAP = 64
_FLUSH_AT = _SCAP - 16


def _make_phase_a():
    mesh = plsc.VectorSubcoreMesh(core_axis_name="c", subcore_axis_name="s")

    @functools.partial(
        pl.kernel,
        mesh=mesh,
        compiler_params=pltpu.CompilerParams(
            needs_layout_passes=False, use_tc_tiling_on_sc=True),
        out_type=jax.ShapeDtypeStruct((_NIDX, 128), jnp.float32),
        scratch_types=[
            pltpu.VMEM((2048,), jnp.int32),
            pltpu.VMEM((_HCAP,), jnp.int32),
            pltpu.VMEM((_HCAP,), jnp.int32),
            pltpu.VMEM((_EMBED, _BS), jnp.float32),
            pltpu.VMEM((_EMBED, _BS), jnp.float32),
            pltpu.VMEM((_EMBED, 128), jnp.float32),
            pltpu.VMEM((_SCAP, 128), jnp.float32),
            pltpu.VMEM((_SCAP,), jnp.int32),
            pltpu.SMEM((17,), jnp.int32),
            pltpu.SMEM((1024,), jnp.int32),
            pltpu.SemaphoreType.DMA,
            pltpu.SemaphoreType.DMA,
            pltpu.SemaphoreType.DMA,
        ],
    )
    def phase_a(idx_hbm, tablet_hbm, tail_hbm, rows_hbm,
                idxc, hits, bkt, tb0, tb1, tail_v, big, posr,
                sm, cnsm, semd0, semd1, semf):
        wid = lax.axis_index("s") * 2 + lax.axis_index("c")
        jlo = wid * _NBW
        lo = jlo * _BS
        hi = jnp.minimum(lo + _NBW * _BS, _VOCAB)
        iota16 = lax.iota(jnp.int32, 16)
        evs = [iota16 + 16 * k for k in range(4)]

        pltpu.sync_copy(tail_hbm, tail_v)

        def compact(nv, maskfn, valfn, dst, cursor):
            nsub = (nv + 1023) >> 10

            def sub(si, cur):
                vbase = si * 1024
                nvh = jnp.minimum(1024, nv - vbase)

                def p1(vv, _):
                    cnt = plsc.all_reduce_population_count(
                        maskfn(vbase + vv))
                    cnsm[vv] = jnp.max(cnt)
                    return 0

                lax.fori_loop(0, nvh, p1, 0)

                def p2(vv, cur):
                    v = vbase + vv
                    plsc.store_compressed(dst.at[pl.ds(cur, 16)], valfn(v),
                                          mask=maskfn(v))
                    return cur + cnsm[vv]

                return lax.fori_loop(0, nvh, p2, cur)

            return lax.fori_loop(0, nsub, sub, cursor)

        nh = jnp.int32(0)
        for c in range(_NIDX // 2048):
            pltpu.sync_copy(idx_hbm.at[pl.ds(c * 2048, 2048)], idxc)

            def maskfn(v):
                r = idxc[pl.ds(v * 16, 16)]
                return (r >= lo) & (r < hi)

            def valfn(v, c=c):
                r = idxc[pl.ds(v * 16, 16)]
                pos = (c * 2048 + v * 16) + iota16
                return (pos << 15) | (r - lo)

            nh = compact(128, maskfn, valfn, hits, nh)

        nv = (nh + 15) >> 4
        cur = jnp.int32(0)
        for b in range(16):
            sm[b] = cur

            def maskfn(v, b=b):
                h = hits[pl.ds(v * 16, 16)]
                valid = (v * 16 + iota16) < nh
                return valid & (((h & 0x7FFF) >> 11) == b)

            def valfn(v):
                return hits[pl.ds(v * 16, 16)]

            cur = compact(nv, maskfn, valfn, bkt, cur)
        sm[16] = cur

        match = hits

        bufs = [tb0, tb1]
        sems = [semd0, semd1]

        def fire(slot, j):
            jc = jnp.minimum(j, _NBF - 1)
            off = pl.multiple_of(jc * _BS, _BS)
            return pltpu.async_copy(tablet_hbm.at[:, pl.ds(off, _BS)],
                                    bufs[slot], sems[slot])

        def flush():
            pltpu.async_copy(
                big, rows_hbm.at[plsc.Indices(posr, ignored_value=-1)],
                semf).wait()
            neg = jnp.full((16,), -1, jnp.int32)
            for q in range(_SCAP // 16):
                posr[pl.ds(q * 16, 16)] = neg

        def process_block(j, jok, src, cursor):
            jrel = j - jlo
            b = jrel >> 3
            s = sm[b]
            t = sm[b + 1]
            v0 = s >> 4
            nv = ((t + 15) >> 4) - v0

            def mmask(vv):
                v = v0 + vv
                h = bkt[pl.ds(v * 16, 16)]
                k = v * 16 + iota16
                return jok & (k >= s) & (k < t) & (
                    ((h & 0x7FFF) >> 8) == jrel)

            def mval(vv):
                return bkt[pl.ds((v0 + vv) * 16, 16)]

            nm = compact(nv, mmask, mval, match, jnp.int32(0))

            def ebody(g, cur):
                cur = lax.cond(cur > _FLUSH_AT,
                               lambda: (flush(), jnp.int32(0))[1],
                               lambda: cur)
                h = match[pl.ds(g * 16, 16)]
                valid = (g * 16 + iota16) < nm
                posr[pl.ds(cur, 16)] = jnp.where(valid, h >> 15, -1)
                nmg = jnp.minimum(16, nm - g * 16)

                def hbody(i, _, g=g):
                    hsp = plsc.load_gather(
                        match, [jnp.full((16,), 0, jnp.int32) + (g * 16 + i)])
                    lane = hsp & 255
                    for k in range(4):
                        vals = plsc.load_gather(src, [evs[k], lane])
                        big[cur + i, pl.ds(k * 16, 16)] = vals
                    return 0

                lax.fori_loop(0, nmg, hbody, 0)
                return cur + 16

            return lax.fori_loop(0, (nm + 15) >> 4, ebody, cursor)

        neg = jnp.full((16,), -1, jnp.int32)
        for q in range(_SCAP // 16):
            posr[pl.ds(q * 16, 16)] = neg

        fire(0, jlo)
        fire(1, jlo + 1)

        def pair(ii, cursor):
            j0 = jlo + 2 * ii
            pltpu.make_async_copy(
                tablet_hbm.at[:, pl.ds(pl.multiple_of(0, _BS), _BS)],
                tb0, semd0).wait()
            cursor = process_block(j0, j0 < _NBF, tb0, cursor)
            fire(0, j0 + 2)
            j1 = j0 + 1
            pltpu.make_async_copy(
                tablet_hbm.at[:, pl.ds(pl.multiple_of(0, _BS), _BS)],
                tb1, semd1).wait()
            cursor = process_block(j1, j1 < _NBF, tb1, cursor)
            fire(1, j1 + 2)
            return cursor

        cursor = lax.fori_loop(0, _PAIRS, pair, jnp.int32(0))

        cursor = lax.cond(wid == _NW - 1,
                          lambda c: process_block(jnp.int32(_NBF), True,
                                                  tail_v, c),
                          lambda c: c, cursor)

        flush()

        dummy = tablet_hbm.at[:, pl.ds(pl.multiple_of(0, _BS), _BS)]
        pltpu.make_async_copy(dummy, tb0, semd0).wait()
        pltpu.make_async_copy(dummy, tb1, semd1).wait()

    return phase_a


def _phase_b_body(wref, cref, wscal, bscal, oref):
    s = jnp.sum((wref[...] * cref[...])[:, :_EMBED], axis=1, keepdims=True)
    z = s * wscal[0, 0] + bscal[0, 0]
    oref[...] = 1.0 / (1.0 + jnp.exp(-z))


def _make_phase_b():
    blk = 256
    grid = _BATCH // blk
    return pl.pallas_call(
        _phase_b_body,
        grid=(grid,),
        in_specs=[
            pl.BlockSpec((blk, 128), lambda i: (i, 0)),
            pl.BlockSpec((blk, 128), lambda i: (i + grid, 0)),
            pl.BlockSpec((1, 1), lambda i: (0, 0), memory_space=pltpu.SMEM),
            pl.BlockSpec((1, 1), lambda i: (0, 0), memory_space=pltpu.SMEM),
        ],
        out_specs=pl.BlockSpec((blk, 1), lambda i: (i, 0)),
        out_shape=jax.ShapeDtypeStruct((_BATCH, 1), jnp.float32),
        compiler_params=pltpu.CompilerParams(
            dimension_semantics=("arbitrary",)),
    )


_phase_a = _make_phase_a()
_phase_b = _make_phase_b()


@jax.jit
def kernel(word, context, table, dense_w, dense_b):
    idx = jnp.concatenate([word.reshape(_BATCH), context.reshape(_BATCH)])
    idx = idx.astype(jnp.int32)
    tablet = table.T
    tail = jnp.pad(table[_TAIL0:].T.astype(jnp.float32), ((0, 0), (0, 64)))
    rows = _phase_a(idx, tablet, tail)
    out = _phase_b(rows, rows,
                   dense_w.reshape(1, 1).astype(jnp.float32),
                   dense_b.reshape(1, 1).astype(jnp.float32))
    return out


# speedup vs baseline: 2.4605x; 1.0017x over previous
import functools

import jax
import jax.numpy as jnp
from jax import lax
from jax.experimental import pallas as pl
from jax.experimental.pallas import tpu as pltpu
from jax.experimental.pallas import tpu_sc as plsc

_VOCAB = 1000000
_EMBED = 64
_BATCH = 16384
_NW = 32
_BS = 256
_NBF = 3906
_TAIL0 = _NBF * _BS
_NBW = 123
_PAIRS = (_NBW + 1) // 2
_NIDX = 2 * _BATCH
_HCAP = _NIDX + 16
_SCAP = 128
_FLUSH_AT = _SCAP - 16


def _make_phase_a():
    mesh = plsc.VectorSubcoreMesh(core_axis_name="c", subcore_axis_name="s")

    @functools.partial(
        pl.kernel,
        mesh=mesh,
        compiler_params=pltpu.CompilerParams(
            needs_layout_passes=False, use_tc_tiling_on_sc=True),
        out_type=jax.ShapeDtypeStruct((_NIDX, 128), jnp.float32),
        scratch_types=[
            pltpu.VMEM((2048,), jnp.int32),
            pltpu.VMEM((_HCAP,), jnp.int32),
            pltpu.VMEM((_HCAP,), jnp.int32),
            pltpu.VMEM((_EMBED, _BS), jnp.float32),
            pltpu.VMEM((_EMBED, _BS), jnp.float32),
            pltpu.VMEM((_EMBED, 128), jnp.float32),
            pltpu.VMEM((_SCAP, 128), jnp.float32),
            pltpu.VMEM((_SCAP,), jnp.int32),
            pltpu.SMEM((17,), jnp.int32),
            pltpu.SMEM((1024,), jnp.int32),
            pltpu.SemaphoreType.DMA,
            pltpu.SemaphoreType.DMA,
            pltpu.SemaphoreType.DMA,
        ],
    )
    def phase_a(idx_hbm, tablet_hbm, tail_hbm, rows_hbm,
                idxc, hits, bkt, tb0, tb1, tail_v, big, posr,
                sm, cnsm, semd0, semd1, semf):
        wid = lax.axis_index("s") * 2 + lax.axis_index("c")
        jlo = wid * _NBW
        lo = jlo * _BS
        hi = jnp.minimum(lo + _NBW * _BS, _VOCAB)
        iota16 = lax.iota(jnp.int32, 16)
        evs = [iota16 + 16 * k for k in range(4)]

        pltpu.sync_copy(tail_hbm, tail_v)

        def compact(nv, maskfn, valfn, dst, cursor):
            nsub = (nv + 1023) >> 10

            def sub(si, cur):
                vbase = si * 1024
                nvh = jnp.minimum(1024, nv - vbase)

                def p1(vv, _):
                    cnt = plsc.all_reduce_population_count(
                        maskfn(vbase + vv))
                    cnsm[vv] = jnp.max(cnt)
                    return 0

                lax.fori_loop(0, nvh, p1, 0)

                def p2(vv, cur):
                    v = vbase + vv
                    plsc.store_compressed(dst.at[pl.ds(cur, 16)], valfn(v),
                                          mask=maskfn(v))
                    return cur + cnsm[vv]

                return lax.fori_loop(0, nvh, p2, cur)

            return lax.fori_loop(0, nsub, sub, cursor)

        nh = jnp.int32(0)
        for c in range(_NIDX // 2048):
            pltpu.sync_copy(idx_hbm.at[pl.ds(c * 2048, 2048)], idxc)

            def maskfn(v):
                r = idxc[pl.ds(v * 16, 16)]
                return (r >= lo) & (r < hi)

            def valfn(v, c=c):
                r = idxc[pl.ds(v * 16, 16)]
                pos = (c * 2048 + v * 16) + iota16
                return (pos << 15) | (r - lo)

            nh = compact(128, maskfn, valfn, hits, nh)

        nv = (nh + 15) >> 4
        cur = jnp.int32(0)
        for b in range(16):
            sm[b] = cur

            def maskfn(v, b=b):
                h = hits[pl.ds(v * 16, 16)]
                valid = (v * 16 + iota16) < nh
                return valid & (((h & 0x7FFF) >> 11) == b)

            def valfn(v):
                return hits[pl.ds(v * 16, 16)]

            cur = compact(nv, maskfn, valfn, bkt, cur)
        sm[16] = cur

        match = hits

        bufs = [tb0, tb1]
        sems = [semd0, semd1]

        def fire(slot, j):
            jc = jnp.minimum(j, _NBF - 1)
            off = pl.multiple_of(jc * _BS, _BS)
            return pltpu.async_copy(tablet_hbm.at[:, pl.ds(off, _BS)],
                                    bufs[slot], sems[slot])

        def flush():
            pltpu.async_copy(
                big, rows_hbm.at[plsc.Indices(posr, ignored_value=-1)],
                semf).wait()
            neg = jnp.full((16,), -1, jnp.int32)
            for q in range(_SCAP // 16):
                posr[pl.ds(q * 16, 16)] = neg

        def process_block(j, jok, src, cursor):
            jrel = j - jlo
            b = jrel >> 3
            s = sm[b]
            t = sm[b + 1]
            v0 = s >> 4
            nv = ((t + 15) >> 4) - v0

            def mmask(vv):
                v = v0 + vv
                h = bkt[pl.ds(v * 16, 16)]
                k = v * 16 + iota16
                return jok & (k >= s) & (k < t) & (
                    ((h & 0x7FFF) >> 8) == jrel)

            def mval(vv):
                return bkt[pl.ds((v0 + vv) * 16, 16)]

            nm = compact(nv, mmask, mval, match, jnp.int32(0))

            def ebody(g, cur):
                cur = lax.cond(cur > _FLUSH_AT,
                               lambda: (flush(), jnp.int32(0))[1],
                               lambda: cur)
                h = match[pl.ds(g * 16, 16)]
                valid = (g * 16 + iota16) < nm
                posr[pl.ds(cur, 16)] = jnp.where(valid, h >> 15, -1)
                nmg = jnp.minimum(16, nm - g * 16)

                def hbody(i, _, g=g):
                    hsp = plsc.load_gather(
                        match, [jnp.full((16,), 0, jnp.int32) + (g * 16 + i)])
                    lane = hsp & 255
                    for k in range(4):
                        vals = plsc.load_gather(src, [evs[k], lane])
                        big[cur + i, pl.ds(k * 16, 16)] = vals
                    return 0

                lax.fori_loop(0, nmg, hbody, 0)
                return cur + 16

            return lax.fori_loop(0, (nm + 15) >> 4, ebody, cursor)

        neg = jnp.full((16,), -1, jnp.int32)
        for q in range(_SCAP // 16):
            posr[pl.ds(q * 16, 16)] = neg

        fire(0, jlo)
        fire(1, jlo + 1)

        def pair(ii, cursor):
            j0 = jlo + 2 * ii
            pltpu.make_async_copy(
                tablet_hbm.at[:, pl.ds(pl.multiple_of(0, _BS), _BS)],
                tb0, semd0).wait()
            cursor = process_block(j0, j0 < _NBF, tb0, cursor)
            fire(0, j0 + 2)
            j1 = j0 + 1
            pltpu.make_async_copy(
                tablet_hbm.at[:, pl.ds(pl.multiple_of(0, _BS), _BS)],
                tb1, semd1).wait()
            cursor = process_block(j1, j1 < _NBF, tb1, cursor)
            fire(1, j1 + 2)
            return cursor

        cursor = lax.fori_loop(0, _PAIRS, pair, jnp.int32(0))

        cursor = lax.cond(wid == _NW - 1,
                          lambda c: process_block(jnp.int32(_NBF), True,
                                                  tail_v, c),
                          lambda c: c, cursor)

        flush()

        dummy = tablet_hbm.at[:, pl.ds(pl.multiple_of(0, _BS), _BS)]
        pltpu.make_async_copy(dummy, tb0, semd0).wait()
        pltpu.make_async_copy(dummy, tb1, semd1).wait()

    return phase_a


def _phase_b_body(wref, cref, wscal, bscal, oref):
    s = jnp.sum((wref[...] * cref[...])[:, :_EMBED], axis=1, keepdims=True)
    z = s * wscal[0, 0] + bscal[0, 0]
    oref[...] = 1.0 / (1.0 + jnp.exp(-z))


def _make_phase_b():
    blk = 256
    grid = _BATCH // blk
    return pl.pallas_call(
        _phase_b_body,
        grid=(grid,),
        in_specs=[
            pl.BlockSpec((blk, 128), lambda i: (i, 0)),
            pl.BlockSpec((blk, 128), lambda i: (i + grid, 0)),
            pl.BlockSpec((1, 1), lambda i: (0, 0), memory_space=pltpu.SMEM),
            pl.BlockSpec((1, 1), lambda i: (0, 0), memory_space=pltpu.SMEM),
        ],
        out_specs=pl.BlockSpec((blk, 1), lambda i: (i, 0)),
        out_shape=jax.ShapeDtypeStruct((_BATCH, 1), jnp.float32),
        compiler_params=pltpu.CompilerParams(
            dimension_semantics=("arbitrary",)),
    )


_phase_a = _make_phase_a()
_phase_b = _make_phase_b()


@jax.jit
def kernel(word, context, table, dense_w, dense_b):
    idx = jnp.concatenate([word.reshape(_BATCH), context.reshape(_BATCH)])
    idx = idx.astype(jnp.int32)
    tablet = table.T
    tail = jnp.pad(table[_TAIL0:].T.astype(jnp.float32), ((0, 0), (0, 64)))
    rows = _phase_a(idx, tablet, tail)
    out = _phase_b(rows, rows,
                   dense_w.reshape(1, 1).astype(jnp.float32),
                   dense_b.reshape(1, 1).astype(jnp.float32))
    return out


# speedup vs baseline: 2.6130x; 1.0620x over previous
import functools

import jax
import jax.numpy as jnp
from jax import lax
from jax.experimental import pallas as pl
from jax.experimental.pallas import tpu as pltpu
from jax.experimental.pallas import tpu_sc as plsc

_VOCAB = 1000000
_EMBED = 64
_BATCH = 16384
_NW = 32
_BS = 128
_NBF = 7812
_TAIL0 = _NBF * _BS
_NBW = 245
_QUADS = (_NBW + 3) // 4
_NIDX = 2 * _BATCH
_HCAP = _NIDX + 16
_SCAP = 128
_FLUSH_AT = _SCAP - 16


def _make_phase_a():
    mesh = plsc.VectorSubcoreMesh(core_axis_name="c", subcore_axis_name="s")

    @functools.partial(
        pl.kernel,
        mesh=mesh,
        compiler_params=pltpu.CompilerParams(
            needs_layout_passes=False, use_tc_tiling_on_sc=True),
        out_type=jax.ShapeDtypeStruct((_NIDX, 128), jnp.float32),
        scratch_types=[
            pltpu.VMEM((2048,), jnp.int32),
            pltpu.VMEM((_HCAP,), jnp.int32),
            pltpu.VMEM((_HCAP,), jnp.int32),
            pltpu.VMEM((_EMBED, _BS), jnp.float32),
            pltpu.VMEM((_EMBED, _BS), jnp.float32),
            pltpu.VMEM((_EMBED, _BS), jnp.float32),
            pltpu.VMEM((_EMBED, _BS), jnp.float32),
            pltpu.VMEM((_EMBED, 128), jnp.float32),
            pltpu.VMEM((_SCAP, 128), jnp.float32),
            pltpu.VMEM((_SCAP,), jnp.int32),
            pltpu.SMEM((17,), jnp.int32),
            pltpu.SMEM((1024,), jnp.int32),
            pltpu.SemaphoreType.DMA,
            pltpu.SemaphoreType.DMA,
            pltpu.SemaphoreType.DMA,
            pltpu.SemaphoreType.DMA,
            pltpu.SemaphoreType.DMA,
        ],
    )
    def phase_a(idx_hbm, tablet_hbm, tail_hbm, rows_hbm,
                idxc, hits, bkt, tb0, tb1, tb2, tb3, tail_v, big, posr,
                sm, cnsm, semd0, semd1, semd2, semd3, semf):
        wid = lax.axis_index("s") * 2 + lax.axis_index("c")
        jlo = wid * _NBW
        lo = jlo * _BS
        hi = jnp.minimum(lo + _NBW * _BS, _VOCAB)
        iota16 = lax.iota(jnp.int32, 16)
        evs = [iota16 + 16 * k for k in range(4)]

        pltpu.sync_copy(tail_hbm, tail_v)

        def compact(nv, maskfn, valfn, dst, cursor):
            nsub = (nv + 1023) >> 10

            def sub(si, cur):
                vbase = si * 1024
                nvh = jnp.minimum(1024, nv - vbase)

                def p1(vv, _):
                    cnt = plsc.all_reduce_population_count(
                        maskfn(vbase + vv))
                    cnsm[vv] = jnp.max(cnt)
                    return 0

                lax.fori_loop(0, nvh, p1, 0)

                def p2(vv, cur):
                    v = vbase + vv
                    plsc.store_compressed(dst.at[pl.ds(cur, 16)], valfn(v),
                                          mask=maskfn(v))
                    return cur + cnsm[vv]

                return lax.fori_loop(0, nvh, p2, cur)

            return lax.fori_loop(0, nsub, sub, cursor)

        nh = jnp.int32(0)
        for c in range(_NIDX // 2048):
            pltpu.sync_copy(idx_hbm.at[pl.ds(c * 2048, 2048)], idxc)

            def maskfn(v):
                r = idxc[pl.ds(v * 16, 16)]
                return (r >= lo) & (r < hi)

            def valfn(v, c=c):
                r = idxc[pl.ds(v * 16, 16)]
                pos = (c * 2048 + v * 16) + iota16
                return (pos << 15) | (r - lo)

            nh = compact(128, maskfn, valfn, hits, nh)

        nv = (nh + 15) >> 4
        cur = jnp.int32(0)
        for b in range(16):
            sm[b] = cur

            def maskfn(v, b=b):
                h = hits[pl.ds(v * 16, 16)]
                valid = (v * 16 + iota16) < nh
                return valid & (((h & 0x7FFF) >> 11) == b)

            def valfn(v):
                return hits[pl.ds(v * 16, 16)]

            cur = compact(nv, maskfn, valfn, bkt, cur)
        sm[16] = cur

        match = hits

        bufs = [tb0, tb1, tb2, tb3]
        sems = [semd0, semd1, semd2, semd3]

        def fire(slot, j):
            jc = jnp.minimum(j, _NBF - 1)
            off = pl.multiple_of(jc * _BS, _BS)
            return pltpu.async_copy(tablet_hbm.at[:, pl.ds(off, _BS)],
                                    bufs[slot], sems[slot])

        def flush():
            pltpu.async_copy(
                big, rows_hbm.at[plsc.Indices(posr, ignored_value=-1)],
                semf).wait()
            neg = jnp.full((16,), -1, jnp.int32)
            for q in range(_SCAP // 16):
                posr[pl.ds(q * 16, 16)] = neg

        def process_block(j, jok, src, cursor):
            jrel = j - jlo
            b = jrel >> 4
            s = sm[b]
            t = sm[b + 1]
            v0 = s >> 4
            nv = ((t + 15) >> 4) - v0

            def mmask(vv):
                v = v0 + vv
                h = bkt[pl.ds(v * 16, 16)]
                k = v * 16 + iota16
                return jok & (k >= s) & (k < t) & (
                    ((h & 0x7FFF) >> 7) == jrel)

            def mval(vv):
                return bkt[pl.ds((v0 + vv) * 16, 16)]

            nm = compact(nv, mmask, mval, match, jnp.int32(0))

            def ebody(g, cur):
                cur = lax.cond(cur > _FLUSH_AT,
                               lambda: (flush(), jnp.int32(0))[1],
                               lambda: cur)
                h = match[pl.ds(g * 16, 16)]
                valid = (g * 16 + iota16) < nm
                posr[pl.ds(cur, 16)] = jnp.where(valid, h >> 15, -1)
                nmg = jnp.minimum(16, nm - g * 16)

                def hbody(i, _, g=g):
                    hsp = plsc.load_gather(
                        match, [jnp.full((16,), 0, jnp.int32) + (g * 16 + i)])
                    lane = hsp & 127
                    for k in range(4):
                        vals = plsc.load_gather(src, [evs[k], lane])
                        big[cur + i, pl.ds(k * 16, 16)] = vals
                    return 0

                lax.fori_loop(0, nmg, hbody, 0)
                return cur + 16

            return lax.fori_loop(0, (nm + 15) >> 4, ebody, cursor)

        neg = jnp.full((16,), -1, jnp.int32)
        for q in range(_SCAP // 16):
            posr[pl.ds(q * 16, 16)] = neg

        for s4 in range(4):
            fire(s4, jlo + s4)

        def quad(ii, cursor):
            j0 = jlo + 4 * ii
            for s4 in range(4):
                pltpu.make_async_copy(
                    tablet_hbm.at[:, pl.ds(pl.multiple_of(0, _BS), _BS)],
                    bufs[s4], sems[s4]).wait()
                cursor = process_block(j0 + s4, (j0 + s4) < _NBF,
                                       bufs[s4], cursor)
                fire(s4, j0 + s4 + 4)
            return cursor

        cursor = lax.fori_loop(0, _QUADS, quad, jnp.int32(0))

        cursor = lax.cond(wid == _NW - 1,
                          lambda c: process_block(jnp.int32(_NBF), True,
                                                  tail_v, c),
                          lambda c: c, cursor)

        flush()

        dummy = tablet_hbm.at[:, pl.ds(pl.multiple_of(0, _BS), _BS)]
        for s4 in range(4):
            pltpu.make_async_copy(dummy, bufs[s4], sems[s4]).wait()

    return phase_a


def _phase_b_body(wref, cref, wscal, bscal, oref):
    s = jnp.sum((wref[...] * cref[...])[:, :_EMBED], axis=1, keepdims=True)
    z = s * wscal[0, 0] + bscal[0, 0]
    oref[...] = 1.0 / (1.0 + jnp.exp(-z))


def _make_phase_b():
    blk = 256
    grid = _BATCH // blk
    return pl.pallas_call(
        _phase_b_body,
        grid=(grid,),
        in_specs=[
            pl.BlockSpec((blk, 128), lambda i: (i, 0)),
            pl.BlockSpec((blk, 128), lambda i: (i + grid, 0)),
            pl.BlockSpec((1, 1), lambda i: (0, 0), memory_space=pltpu.SMEM),
            pl.BlockSpec((1, 1), lambda i: (0, 0), memory_space=pltpu.SMEM),
        ],
        out_specs=pl.BlockSpec((blk, 1), lambda i: (i, 0)),
        out_shape=jax.ShapeDtypeStruct((_BATCH, 1), jnp.float32),
        compiler_params=pltpu.CompilerParams(
            dimension_semantics=("arbitrary",)),
    )


_phase_a = _make_phase_a()
_phase_b = _make_phase_b()


@jax.jit
def kernel(word, context, table, dense_w, dense_b):
    idx = jnp.concatenate([word.reshape(_BATCH), context.reshape(_BATCH)])
    idx = idx.astype(jnp.int32)
    tablet = table.T
    tail = jnp.pad(table[_TAIL0:].T.astype(jnp.float32), ((0, 0), (0, 64)))
    rows = _phase_a(idx, tablet, tail)
    out = _phase_b(rows, rows,
                   dense_w.reshape(1, 1).astype(jnp.float32),
                   dense_b.reshape(1, 1).astype(jnp.float32))
    return out


# speedup vs baseline: 2.6697x; 1.0217x over previous
import functools

import jax
import jax.numpy as jnp
from jax import lax
from jax.experimental import pallas as pl
from jax.experimental.pallas import tpu as pltpu
from jax.experimental.pallas import tpu_sc as plsc

_VOCAB = 1000000
_EMBED = 64
_BATCH = 16384
_NW = 32
_BS = 128
_NBF = 7812
_TAIL0 = _NBF * _BS
_NBW = 245
_NSLOT = 5
_QUADS = (_NBW + _NSLOT - 1) // _NSLOT
_NIDX = 2 * _BATCH
_HCAP = _NIDX + 16
_SCAP = 64
_FLUSH_AT = _SCAP - 16


def _make_phase_a():
    mesh = plsc.VectorSubcoreMesh(core_axis_name="c", subcore_axis_name="s")

    @functools.partial(
        pl.kernel,
        mesh=mesh,
        compiler_params=pltpu.CompilerParams(
            needs_layout_passes=False, use_tc_tiling_on_sc=True),
        out_type=jax.ShapeDtypeStruct((_NIDX, 128), jnp.float32),
        scratch_types=[
            pltpu.VMEM((2048,), jnp.int32),
            pltpu.VMEM((_HCAP,), jnp.int32),
            pltpu.VMEM((_HCAP,), jnp.int32),
            pltpu.VMEM((_EMBED, _BS), jnp.float32),
            pltpu.VMEM((_EMBED, _BS), jnp.float32),
            pltpu.VMEM((_EMBED, _BS), jnp.float32),
            pltpu.VMEM((_EMBED, _BS), jnp.float32),
            pltpu.VMEM((_EMBED, _BS), jnp.float32),
            pltpu.VMEM((_EMBED, 128), jnp.float32),
            pltpu.VMEM((_SCAP, 128), jnp.float32),
            pltpu.VMEM((_SCAP,), jnp.int32),
            pltpu.SMEM((17,), jnp.int32),
            pltpu.SMEM((1024,), jnp.int32),
            pltpu.SemaphoreType.DMA,
            pltpu.SemaphoreType.DMA,
            pltpu.SemaphoreType.DMA,
            pltpu.SemaphoreType.DMA,
            pltpu.SemaphoreType.DMA,
            pltpu.SemaphoreType.DMA,
        ],
    )
    def phase_a(idx_hbm, tablet_hbm, tail_hbm, rows_hbm,
                idxc, hits, bkt, tb0, tb1, tb2, tb3, tb4, tail_v,
                big, posr, sm, cnsm,
                semd0, semd1, semd2, semd3, semd4, semf):
        wid = lax.axis_index("s") * 2 + lax.axis_index("c")
        jlo = wid * _NBW
        lo = jlo * _BS
        hi = jnp.minimum(lo + _NBW * _BS, _VOCAB)
        iota16 = lax.iota(jnp.int32, 16)
        evs = [iota16 + 16 * k for k in range(4)]

        pltpu.sync_copy(tail_hbm, tail_v)

        def compact(nv, maskfn, valfn, dst, cursor):
            nsub = (nv + 1023) >> 10

            def sub(si, cur):
                vbase = si * 1024
                nvh = jnp.minimum(1024, nv - vbase)

                def p1(vv, _):
                    cnt = plsc.all_reduce_population_count(
                        maskfn(vbase + vv))
                    cnsm[vv] = jnp.max(cnt)
                    return 0

                lax.fori_loop(0, nvh, p1, 0)

                def p2(vv, cur):
                    v = vbase + vv
                    plsc.store_compressed(dst.at[pl.ds(cur, 16)], valfn(v),
                                          mask=maskfn(v))
                    return cur + cnsm[vv]

                return lax.fori_loop(0, nvh, p2, cur)

            return lax.fori_loop(0, nsub, sub, cursor)

        nh = jnp.int32(0)
        for c in range(_NIDX // 2048):
            pltpu.sync_copy(idx_hbm.at[pl.ds(c * 2048, 2048)], idxc)

            def maskfn(v):
                r = idxc[pl.ds(v * 16, 16)]
                return (r >= lo) & (r < hi)

            def valfn(v, c=c):
                r = idxc[pl.ds(v * 16, 16)]
                pos = (c * 2048 + v * 16) + iota16
                return (pos << 15) | (r - lo)

            nh = compact(128, maskfn, valfn, hits, nh)

        nv = (nh + 15) >> 4
        cur = jnp.int32(0)
        for b in range(16):
            sm[b] = cur

            def maskfn(v, b=b):
                h = hits[pl.ds(v * 16, 16)]
                valid = (v * 16 + iota16) < nh
                return valid & (((h & 0x7FFF) >> 11) == b)

            def valfn(v):
                return hits[pl.ds(v * 16, 16)]

            cur = compact(nv, maskfn, valfn, bkt, cur)
        sm[16] = cur

        match = hits

        bufs = [tb0, tb1, tb2, tb3, tb4]
        sems = [semd0, semd1, semd2, semd3, semd4]

        def fire(slot, j):
            jc = jnp.minimum(j, _NBF - 1)
            off = pl.multiple_of(jc * _BS, _BS)
            return pltpu.async_copy(tablet_hbm.at[:, pl.ds(off, _BS)],
                                    bufs[slot], sems[slot])

        def flush():
            pltpu.async_copy(
                big, rows_hbm.at[plsc.Indices(posr, ignored_value=-1)],
                semf).wait()
            neg = jnp.full((16,), -1, jnp.int32)
            for q in range(_SCAP // 16):
                posr[pl.ds(q * 16, 16)] = neg

        def process_block(j, jok, src, cursor):
            jrel = j - jlo
            b = jrel >> 4
            s = sm[b]
            t = sm[b + 1]
            v0 = s >> 4
            nv = ((t + 15) >> 4) - v0

            def mmask(vv):
                v = v0 + vv
                h = bkt[pl.ds(v * 16, 16)]
                k = v * 16 + iota16
                return jok & (k >= s) & (k < t) & (
                    ((h & 0x7FFF) >> 7) == jrel)

            def mval(vv):
                return bkt[pl.ds((v0 + vv) * 16, 16)]

            nm = compact(nv, mmask, mval, match, jnp.int32(0))

            def ebody(g, cur):
                cur = lax.cond(cur > _FLUSH_AT,
                               lambda: (flush(), jnp.int32(0))[1],
                               lambda: cur)
                h = match[pl.ds(g * 16, 16)]
                valid = (g * 16 + iota16) < nm
                posr[pl.ds(cur, 16)] = jnp.where(valid, h >> 15, -1)
                nmg = jnp.minimum(16, nm - g * 16)

                def hbody(i, _, g=g):
                    hsp = plsc.load_gather(
                        match, [jnp.full((16,), 0, jnp.int32) + (g * 16 + i)])
                    lane = hsp & 127
                    for k in range(4):
                        vals = plsc.load_gather(src, [evs[k], lane])
                        big[cur + i, pl.ds(k * 16, 16)] = vals
                    return 0

                lax.fori_loop(0, nmg, hbody, 0)
                return cur + 16

            return lax.fori_loop(0, (nm + 15) >> 4, ebody, cursor)

        neg = jnp.full((16,), -1, jnp.int32)
        for q in range(_SCAP // 16):
            posr[pl.ds(q * 16, 16)] = neg

        for s4 in range(_NSLOT):
            fire(s4, jlo + s4)

        def quad(ii, cursor):
            j0 = jlo + _NSLOT * ii
            for s4 in range(_NSLOT):
                pltpu.make_async_copy(
                    tablet_hbm.at[:, pl.ds(pl.multiple_of(0, _BS), _BS)],
                    bufs[s4], sems[s4]).wait()
                cursor = process_block(j0 + s4, (j0 + s4) < _NBF,
                                       bufs[s4], cursor)
                fire(s4, j0 + s4 + _NSLOT)
            return cursor

        cursor = lax.fori_loop(0, _QUADS, quad, jnp.int32(0))

        cursor = lax.cond(wid == _NW - 1,
                          lambda c: process_block(jnp.int32(_NBF), True,
                                                  tail_v, c),
                          lambda c: c, cursor)

        flush()

        dummy = tablet_hbm.at[:, pl.ds(pl.multiple_of(0, _BS), _BS)]
        for s4 in range(_NSLOT):
            pltpu.make_async_copy(dummy, bufs[s4], sems[s4]).wait()

    return phase_a


def _phase_b_body(wref, cref, wscal, bscal, oref):
    s = jnp.sum((wref[...] * cref[...])[:, :_EMBED], axis=1, keepdims=True)
    z = s * wscal[0, 0] + bscal[0, 0]
    oref[...] = 1.0 / (1.0 + jnp.exp(-z))


def _make_phase_b():
    blk = 256
    grid = _BATCH // blk
    return pl.pallas_call(
        _phase_b_body,
        grid=(grid,),
        in_specs=[
            pl.BlockSpec((blk, 128), lambda i: (i, 0)),
            pl.BlockSpec((blk, 128), lambda i: (i + grid, 0)),
            pl.BlockSpec((1, 1), lambda i: (0, 0), memory_space=pltpu.SMEM),
            pl.BlockSpec((1, 1), lambda i: (0, 0), memory_space=pltpu.SMEM),
        ],
        out_specs=pl.BlockSpec((blk, 1), lambda i: (i, 0)),
        out_shape=jax.ShapeDtypeStruct((_BATCH, 1), jnp.float32),
        compiler_params=pltpu.CompilerParams(
            dimension_semantics=("arbitrary",)),
    )


_phase_a = _make_phase_a()
_phase_b = _make_phase_b()


@jax.jit
def kernel(word, context, table, dense_w, dense_b):
    idx = jnp.concatenate([word.reshape(_BATCH), context.reshape(_BATCH)])
    idx = idx.astype(jnp.int32)
    tablet = table.T
    tail = jnp.pad(table[_TAIL0:].T.astype(jnp.float32), ((0, 0), (0, 64)))
    rows = _phase_a(idx, tablet, tail)
    out = _phase_b(rows, rows,
                   dense_w.reshape(1, 1).astype(jnp.float32),
                   dense_b.reshape(1, 1).astype(jnp.float32))
    return out


# speedup vs baseline: 2.9574x; 1.1078x over previous
import functools

import jax
import jax.numpy as jnp
from jax import lax
from jax.experimental import pallas as pl
from jax.experimental.pallas import tpu as pltpu
from jax.experimental.pallas import tpu_sc as plsc

_VOCAB = 1000000
_EMBED = 64
_BATCH = 16384
_NW = 32
_BS = 128
_NBF = 7812
_TAIL0 = _NBF * _BS
_NBW = 245
_NSLOT = 5
_QUADS = (_NBW + _NSLOT - 1) // _NSLOT
_NIDX = 2 * _BATCH
_HCAP = _NIDX + 16
_SCAP = 64
_FLUSH_AT = _SCAP - 16


def _make_phase_a():
    mesh = plsc.VectorSubcoreMesh(core_axis_name="c", subcore_axis_name="s")

    @functools.partial(
        pl.kernel,
        mesh=mesh,
        compiler_params=pltpu.CompilerParams(
            needs_layout_passes=False, use_tc_tiling_on_sc=True),
        out_type=jax.ShapeDtypeStruct((_NIDX, 128), jnp.float32),
        scratch_types=[
            pltpu.VMEM((2048,), jnp.int32),
            pltpu.VMEM((_HCAP,), jnp.int32),
            pltpu.VMEM((_HCAP,), jnp.int32),
            pltpu.VMEM((_EMBED, _BS), jnp.float32),
            pltpu.VMEM((_EMBED, _BS), jnp.float32),
            pltpu.VMEM((_EMBED, _BS), jnp.float32),
            pltpu.VMEM((_EMBED, _BS), jnp.float32),
            pltpu.VMEM((_EMBED, _BS), jnp.float32),
            pltpu.VMEM((_EMBED, 128), jnp.float32),
            pltpu.VMEM((_SCAP, 128), jnp.float32),
            pltpu.VMEM((_SCAP,), jnp.int32),
            pltpu.SMEM((17,), jnp.int32),
            pltpu.SMEM((1024,), jnp.int32),
            pltpu.SemaphoreType.DMA,
            pltpu.SemaphoreType.DMA,
            pltpu.SemaphoreType.DMA,
            pltpu.SemaphoreType.DMA,
            pltpu.SemaphoreType.DMA,
            pltpu.SemaphoreType.DMA,
        ],
    )
    def phase_a(widx_hbm, cidx_hbm, tablet_hbm, tail_hbm, rows_hbm,
                idxc, hits, bkt, tb0, tb1, tb2, tb3, tb4, tail_v,
                big, posr, sm, cnsm,
                semd0, semd1, semd2, semd3, semd4, semf):
        wid = lax.axis_index("s") * 2 + lax.axis_index("c")
        jlo = wid * _NBW
        lo = jlo * _BS
        hi = jnp.minimum(lo + _NBW * _BS, _VOCAB)
        iota16 = lax.iota(jnp.int32, 16)
        evs = [iota16 + 16 * k for k in range(4)]

        pltpu.sync_copy(tail_hbm, tail_v)

        def compact(nv, maskfn, valfn, dst, cursor):
            nsub = (nv + 1023) >> 10

            def sub(si, cur):
                vbase = si * 1024
                nvh = jnp.minimum(1024, nv - vbase)

                def p1(vv, _):
                    cnt = plsc.all_reduce_population_count(
                        maskfn(vbase + vv))
                    cnsm[vv] = jnp.max(cnt)
                    return 0

                lax.fori_loop(0, nvh, p1, 0)

                def p2(vv, cur):
                    v = vbase + vv
                    plsc.store_compressed(dst.at[pl.ds(cur, 16)], valfn(v),
                                          mask=maskfn(v))
                    return cur + cnsm[vv]

                return lax.fori_loop(0, nvh, p2, cur)

            return lax.fori_loop(0, nsub, sub, cursor)

        nh = jnp.int32(0)
        for c in range(_NIDX // 2048):
            src_hbm = widx_hbm if c < _BATCH // 2048 else cidx_hbm
            pltpu.sync_copy(
                src_hbm.at[pl.ds((c * 2048) % _BATCH, 2048)], idxc)

            def maskfn(v):
                r = idxc[pl.ds(v * 16, 16)]
                return (r >= lo) & (r < hi)

            def valfn(v, c=c):
                r = idxc[pl.ds(v * 16, 16)]
                pos = (c * 2048 + v * 16) + iota16
                return (pos << 15) | (r - lo)

            nh = compact(128, maskfn, valfn, hits, nh)

        nv = (nh + 15) >> 4
        cur = jnp.int32(0)
        for b in range(16):
            sm[b] = cur

            def maskfn(v, b=b):
                h = hits[pl.ds(v * 16, 16)]
                valid = (v * 16 + iota16) < nh
                return valid & (((h & 0x7FFF) >> 11) == b)

            def valfn(v):
                return hits[pl.ds(v * 16, 16)]

            cur = compact(nv, maskfn, valfn, bkt, cur)
        sm[16] = cur

        match = hits

        bufs = [tb0, tb1, tb2, tb3, tb4]
        sems = [semd0, semd1, semd2, semd3, semd4]

        def fire(slot, j):
            jc = jnp.minimum(j, _NBF - 1)
            off = pl.multiple_of(jc * _BS, _BS)
            return pltpu.async_copy(tablet_hbm.at[:, pl.ds(off, _BS)],
                                    bufs[slot], sems[slot])

        def flush():
            pltpu.async_copy(
                big, rows_hbm.at[plsc.Indices(posr, ignored_value=-1)],
                semf).wait()
            neg = jnp.full((16,), -1, jnp.int32)
            for q in range(_SCAP // 16):
                posr[pl.ds(q * 16, 16)] = neg

        def process_block(j, jok, src, cursor):
            jrel = j - jlo
            b = jrel >> 4
            s = sm[b]
            t = sm[b + 1]
            v0 = s >> 4
            nv = ((t + 15) >> 4) - v0

            def mmask(vv):
                v = v0 + vv
                h = bkt[pl.ds(v * 16, 16)]
                k = v * 16 + iota16
                return jok & (k >= s) & (k < t) & (
                    ((h & 0x7FFF) >> 7) == jrel)

            def mval(vv):
                return bkt[pl.ds((v0 + vv) * 16, 16)]

            nm = compact(nv, mmask, mval, match, jnp.int32(0))

            def ebody(g, cur):
                cur = lax.cond(cur > _FLUSH_AT,
                               lambda: (flush(), jnp.int32(0))[1],
                               lambda: cur)
                h = match[pl.ds(g * 16, 16)]
                valid = (g * 16 + iota16) < nm
                posr[pl.ds(cur, 16)] = jnp.where(valid, h >> 15, -1)
                nmg = jnp.minimum(16, nm - g * 16)

                def hbody(i, _, g=g):
                    hsp = plsc.load_gather(
                        match, [jnp.full((16,), 0, jnp.int32) + (g * 16 + i)])
                    lane = hsp & 127
                    for k in range(4):
                        vals = plsc.load_gather(src, [evs[k], lane])
                        big[cur + i, pl.ds(k * 16, 16)] = vals
                    return 0

                lax.fori_loop(0, nmg, hbody, 0)
                return cur + 16

            return lax.fori_loop(0, (nm + 15) >> 4, ebody, cursor)

        neg = jnp.full((16,), -1, jnp.int32)
        for q in range(_SCAP // 16):
            posr[pl.ds(q * 16, 16)] = neg

        for s4 in range(_NSLOT):
            fire(s4, jlo + s4)

        def quad(ii, cursor):
            j0 = jlo + _NSLOT * ii
            for s4 in range(_NSLOT):
                pltpu.make_async_copy(
                    tablet_hbm.at[:, pl.ds(pl.multiple_of(0, _BS), _BS)],
                    bufs[s4], sems[s4]).wait()
                cursor = process_block(j0 + s4, (j0 + s4) < _NBF,
                                       bufs[s4], cursor)
                fire(s4, j0 + s4 + _NSLOT)
            return cursor

        cursor = lax.fori_loop(0, _QUADS, quad, jnp.int32(0))

        cursor = lax.cond(wid == _NW - 1,
                          lambda c: process_block(jnp.int32(_NBF), True,
                                                  tail_v, c),
                          lambda c: c, cursor)

        flush()

        dummy = tablet_hbm.at[:, pl.ds(pl.multiple_of(0, _BS), _BS)]
        for s4 in range(_NSLOT):
            pltpu.make_async_copy(dummy, bufs[s4], sems[s4]).wait()

    return phase_a


def _phase_b_body(wref, cref, wscal, bscal, oref):
    s = jnp.sum((wref[...] * cref[...])[:, :_EMBED], axis=1, keepdims=True)
    z = s * wscal[0, 0] + bscal[0, 0]
    oref[...] = 1.0 / (1.0 + jnp.exp(-z))


def _make_phase_b():
    blk = 1024
    grid = _BATCH // blk
    return pl.pallas_call(
        _phase_b_body,
        grid=(grid,),
        in_specs=[
            pl.BlockSpec((blk, 128), lambda i: (i, 0)),
            pl.BlockSpec((blk, 128), lambda i: (i + grid, 0)),
            pl.BlockSpec((1, 1), lambda i: (0, 0), memory_space=pltpu.SMEM),
            pl.BlockSpec((1, 1), lambda i: (0, 0), memory_space=pltpu.SMEM),
        ],
        out_specs=pl.BlockSpec((blk, 1), lambda i: (i, 0)),
        out_shape=jax.ShapeDtypeStruct((_BATCH, 1), jnp.float32),
        compiler_params=pltpu.CompilerParams(
            dimension_semantics=("arbitrary",)),
    )


_phase_a = _make_phase_a()
_phase_b = _make_phase_b()


@jax.jit
def kernel(word, context, table, dense_w, dense_b):
    widx = word.reshape(_BATCH).astype(jnp.int32)
    cidx = context.reshape(_BATCH).astype(jnp.int32)
    tablet = table.T
    tail = jnp.pad(table[_TAIL0:].T.astype(jnp.float32), ((0, 0), (0, 64)))
    rows = _phase_a(widx, cidx, tablet, tail)
    out = _phase_b(rows, rows,
                   dense_w.reshape(1, 1).astype(jnp.float32),
                   dense_b.reshape(1, 1).astype(jnp.float32))
    return out


# speedup vs baseline: 3.0141x; 1.0192x over previous
import functools

import jax
import jax.numpy as jnp
from jax import lax
from jax.experimental import pallas as pl
from jax.experimental.pallas import tpu as pltpu
from jax.experimental.pallas import tpu_sc as plsc

_VOCAB = 1000000
_EMBED = 64
_BATCH = 16384
_NW = 32
_BS = 128
_NBF = 7812
_TAIL0 = _NBF * _BS
_NBW = 245
_NSLOT = 5
_QUADS = (_NBW + _NSLOT - 1) // _NSLOT
_NIDX = 2 * _BATCH
_HCAP = _NIDX + 16
_SCAP = 64
_FLUSH_AT = _SCAP - 16


def _make_phase_a():
    mesh = plsc.VectorSubcoreMesh(core_axis_name="c", subcore_axis_name="s")

    @functools.partial(
        pl.kernel,
        mesh=mesh,
        compiler_params=pltpu.CompilerParams(
            needs_layout_passes=False, use_tc_tiling_on_sc=True),
        out_type=jax.ShapeDtypeStruct((_NIDX, 128), jnp.float32),
        scratch_types=[
            pltpu.VMEM((2048,), jnp.int32),
            pltpu.VMEM((_HCAP,), jnp.int32),
            pltpu.VMEM((_HCAP,), jnp.int32),
            pltpu.VMEM((_EMBED, _BS), jnp.float32),
            pltpu.VMEM((_EMBED, _BS), jnp.float32),
            pltpu.VMEM((_EMBED, _BS), jnp.float32),
            pltpu.VMEM((_EMBED, _BS), jnp.float32),
            pltpu.VMEM((_EMBED, _BS), jnp.float32),
            pltpu.VMEM((_EMBED, 128), jnp.float32),
            pltpu.VMEM((_SCAP, 128), jnp.float32),
            pltpu.VMEM((_SCAP,), jnp.int32),
            pltpu.SMEM((17,), jnp.int32),
            pltpu.SMEM((1024,), jnp.int32),
            pltpu.SemaphoreType.DMA,
            pltpu.SemaphoreType.DMA,
            pltpu.SemaphoreType.DMA,
            pltpu.SemaphoreType.DMA,
            pltpu.SemaphoreType.DMA,
            pltpu.SemaphoreType.DMA,
        ],
    )
    def phase_a(widx_hbm, cidx_hbm, tablet_hbm, tail_hbm, rows_hbm,
                idxc, hits, bkt, tb0, tb1, tb2, tb3, tb4, tail_v,
                big, posr, sm, cnsm,
                semd0, semd1, semd2, semd3, semd4, semf):
        wid = lax.axis_index("s") * 2 + lax.axis_index("c")
        jlo = wid * _NBW
        lo = jlo * _BS
        hi = jnp.minimum(lo + _NBW * _BS, _VOCAB)
        iota16 = lax.iota(jnp.int32, 16)
        evs = [iota16 + 16 * k for k in range(4)]

        pltpu.sync_copy(tail_hbm, tail_v)

        def compact(nv, maskfn, valfn, dst, cursor):
            nsub = (nv + 1023) >> 10

            def sub(si, cur):
                vbase = si * 1024
                nvh = jnp.minimum(1024, nv - vbase)

                def p1(vv, _):
                    cnt = plsc.all_reduce_population_count(
                        maskfn(vbase + vv))
                    cnsm[vv] = jnp.max(cnt)
                    return 0

                lax.fori_loop(0, nvh, p1, 0)

                def p2(vv, cur):
                    v = vbase + vv
                    plsc.store_compressed(dst.at[pl.ds(cur, 16)], valfn(v),
                                          mask=maskfn(v))
                    return cur + cnsm[vv]

                return lax.fori_loop(0, nvh, p2, cur)

            return lax.fori_loop(0, nsub, sub, cursor)

        nh = jnp.int32(0)
        for c in range(_NIDX // 2048):
            src_hbm = widx_hbm if c < _BATCH // 2048 else cidx_hbm
            pltpu.sync_copy(
                src_hbm.at[pl.ds((c * 2048) % _BATCH, 2048)], idxc)

            def maskfn(v):
                r = idxc[pl.ds(v * 16, 16)]
                return (r >= lo) & (r < hi)

            def valfn(v, c=c):
                r = idxc[pl.ds(v * 16, 16)]
                pos = (c * 2048 + v * 16) + iota16
                return (pos << 15) | (r - lo)

            nh = compact(128, maskfn, valfn, hits, nh)

        nv = (nh + 15) >> 4
        cur = jnp.int32(0)
        for b in range(16):
            sm[b] = cur

            def maskfn(v, b=b):
                h = hits[pl.ds(v * 16, 16)]
                valid = (v * 16 + iota16) < nh
                return valid & (((h & 0x7FFF) >> 11) == b)

            def valfn(v):
                return hits[pl.ds(v * 16, 16)]

            cur = compact(nv, maskfn, valfn, bkt, cur)
        sm[16] = cur

        match = hits

        bufs = [tb0, tb1, tb2, tb3, tb4]
        sems = [semd0, semd1, semd2, semd3, semd4]

        def fire(slot, j):
            jc = jnp.minimum(j, _NBF - 1)
            off = pl.multiple_of(jc * _BS, _BS)
            return pltpu.async_copy(tablet_hbm.at[:, pl.ds(off, _BS)],
                                    bufs[slot], sems[slot])

        def flush():
            pltpu.async_copy(
                big, rows_hbm.at[plsc.Indices(posr, ignored_value=-1)],
                semf).wait()
            neg = jnp.full((16,), -1, jnp.int32)
            for q in range(_SCAP // 16):
                posr[pl.ds(q * 16, 16)] = neg

        def process_block(j, jok, src, cursor):
            jrel = j - jlo
            b = jrel >> 4
            s = sm[b]
            t = sm[b + 1]
            v0 = s >> 4
            nv = ((t + 15) >> 4) - v0

            def mmask(vv):
                v = v0 + vv
                h = bkt[pl.ds(v * 16, 16)]
                k = v * 16 + iota16
                return jok & (k >= s) & (k < t) & (
                    ((h & 0x7FFF) >> 7) == jrel)

            def mval(vv):
                return bkt[pl.ds((v0 + vv) * 16, 16)]

            nm = compact(nv, mmask, mval, match, jnp.int32(0))

            def ebody(g, cur):
                cur = lax.cond(cur > _FLUSH_AT,
                               lambda: (flush(), jnp.int32(0))[1],
                               lambda: cur)
                h = match[pl.ds(g * 16, 16)]
                valid = (g * 16 + iota16) < nm
                posr[pl.ds(cur, 16)] = jnp.where(valid, h >> 15, -1)
                nmg = jnp.minimum(16, nm - g * 16)

                def hbody(i, _, g=g):
                    hsp = plsc.load_gather(
                        match, [jnp.full((16,), 0, jnp.int32) + (g * 16 + i)])
                    lane = hsp & 127
                    for k in range(4):
                        vals = plsc.load_gather(src, [evs[k], lane])
                        big[cur + i, pl.ds(k * 16, 16)] = vals
                    return 0

                lax.fori_loop(0, nmg, hbody, 0)
                return cur + 16

            return lax.fori_loop(0, (nm + 15) >> 4, ebody, cursor)

        neg = jnp.full((16,), -1, jnp.int32)
        for q in range(_SCAP // 16):
            posr[pl.ds(q * 16, 16)] = neg

        for s4 in range(_NSLOT):
            fire(s4, jlo + s4)

        def quad(ii, cursor):
            j0 = jlo + _NSLOT * ii
            for s4 in range(_NSLOT):
                pltpu.make_async_copy(
                    tablet_hbm.at[:, pl.ds(pl.multiple_of(0, _BS), _BS)],
                    bufs[s4], sems[s4]).wait()
                cursor = process_block(j0 + s4, (j0 + s4) < _NBF,
                                       bufs[s4], cursor)
                fire(s4, j0 + s4 + _NSLOT)
            return cursor

        cursor = lax.fori_loop(0, _QUADS, quad, jnp.int32(0))

        cursor = lax.cond(wid == _NW - 1,
                          lambda c: process_block(jnp.int32(_NBF), True,
                                                  tail_v, c),
                          lambda c: c, cursor)

        flush()

        dummy = tablet_hbm.at[:, pl.ds(pl.multiple_of(0, _BS), _BS)]
        for s4 in range(_NSLOT):
            pltpu.make_async_copy(dummy, bufs[s4], sems[s4]).wait()

    return phase_a


def _phase_b_body(wref, cref, wscal, bscal, oref):
    s = jnp.sum((wref[...] * cref[...])[:, :_EMBED], axis=1, keepdims=True)
    z = s * wscal[0, 0] + bscal[0, 0]
    oref[...] = 1.0 / (1.0 + jnp.exp(-z))


def _make_phase_b():
    blk = 2048
    grid = _BATCH // blk
    return pl.pallas_call(
        _phase_b_body,
        grid=(grid,),
        in_specs=[
            pl.BlockSpec((blk, 128), lambda i: (i, 0)),
            pl.BlockSpec((blk, 128), lambda i: (i + grid, 0)),
            pl.BlockSpec((1, 1), lambda i: (0, 0), memory_space=pltpu.SMEM),
            pl.BlockSpec((1, 1), lambda i: (0, 0), memory_space=pltpu.SMEM),
        ],
        out_specs=pl.BlockSpec((blk, 1), lambda i: (i, 0)),
        out_shape=jax.ShapeDtypeStruct((_BATCH, 1), jnp.float32),
        compiler_params=pltpu.CompilerParams(
            dimension_semantics=("arbitrary",)),
    )


_phase_a = _make_phase_a()
_phase_b = _make_phase_b()


@jax.jit
def kernel(word, context, table, dense_w, dense_b):
    widx = word.reshape(_BATCH).astype(jnp.int32)
    cidx = context.reshape(_BATCH).astype(jnp.int32)
    tablet = table.T
    tail = jnp.pad(table[_TAIL0:].T.astype(jnp.float32), ((0, 0), (0, 64)))
    rows = _phase_a(widx, cidx, tablet, tail)
    out = _phase_b(rows, rows,
                   dense_w.reshape(1, 1).astype(jnp.float32),
                   dense_b.reshape(1, 1).astype(jnp.float32))
    return out


# speedup vs baseline: 3.0439x; 1.0099x over previous
import functools

import jax
import jax.numpy as jnp
from jax import lax
from jax.experimental import pallas as pl
from jax.experimental.pallas import tpu as pltpu
from jax.experimental.pallas import tpu_sc as plsc

_VOCAB = 1000000
_EMBED = 64
_BATCH = 16384
_NW = 32
_BS = 128
_NBF = 7812
_TAIL0 = _NBF * _BS
_NBW = 245
_NSLOT = 5
_QUADS = (_NBW + _NSLOT - 1) // _NSLOT
_NIDX = 2 * _BATCH
_HCAP = _NIDX + 16
_SCAP = 64
_FLUSH_AT = _SCAP - 16


def _make_phase_a():
    mesh = plsc.VectorSubcoreMesh(core_axis_name="c", subcore_axis_name="s")

    @functools.partial(
        pl.kernel,
        mesh=mesh,
        compiler_params=pltpu.CompilerParams(
            needs_layout_passes=False, use_tc_tiling_on_sc=True),
        out_type=jax.ShapeDtypeStruct((_NIDX, 128), jnp.float32),
        scratch_types=[
            pltpu.VMEM((2048,), jnp.int32),
            pltpu.VMEM((_HCAP,), jnp.int32),
            pltpu.VMEM((_HCAP,), jnp.int32),
            pltpu.VMEM((_EMBED, _BS), jnp.float32),
            pltpu.VMEM((_EMBED, _BS), jnp.float32),
            pltpu.VMEM((_EMBED, _BS), jnp.float32),
            pltpu.VMEM((_EMBED, _BS), jnp.float32),
            pltpu.VMEM((_EMBED, _BS), jnp.float32),
            pltpu.VMEM((_EMBED, 128), jnp.float32),
            pltpu.VMEM((_SCAP, 128), jnp.float32),
            pltpu.VMEM((_SCAP,), jnp.int32),
            pltpu.SMEM((17,), jnp.int32),
            pltpu.SMEM((1024,), jnp.int32),
            pltpu.SemaphoreType.DMA,
            pltpu.SemaphoreType.DMA,
            pltpu.SemaphoreType.DMA,
            pltpu.SemaphoreType.DMA,
            pltpu.SemaphoreType.DMA,
            pltpu.SemaphoreType.DMA,
        ],
    )
    def phase_a(widx_hbm, cidx_hbm, tablet_hbm, tail_hbm, rows_hbm,
                idxc, hits, bkt, tb0, tb1, tb2, tb3, tb4, tail_v,
                big, posr, sm, cnsm,
                semd0, semd1, semd2, semd3, semd4, semf):
        wid = lax.axis_index("s") * 2 + lax.axis_index("c")
        jlo = wid * _NBW
        lo = jlo * _BS
        hi = jnp.minimum(lo + _NBW * _BS, _VOCAB)
        iota16 = lax.iota(jnp.int32, 16)
        evs = [iota16 + 16 * k for k in range(4)]

        pltpu.sync_copy(tail_hbm, tail_v)

        bufs = [tb0, tb1, tb2, tb3, tb4]
        sems = [semd0, semd1, semd2, semd3, semd4]

        def fire(slot, j):
            jc = jnp.minimum(j, _NBF - 1)
            off = pl.multiple_of(jc * _BS, _BS)
            return pltpu.async_copy(tablet_hbm.at[:, pl.ds(off, _BS)],
                                    bufs[slot], sems[slot])

        for s4 in range(_NSLOT):
            fire(s4, jlo + s4)

        def compact(nv, maskfn, valfn, dst, cursor):
            nsub = (nv + 1023) >> 10

            def sub(si, cur):
                vbase = si * 1024
                nvh = jnp.minimum(1024, nv - vbase)

                def p1(vv, _):
                    cnt = plsc.all_reduce_population_count(
                        maskfn(vbase + vv))
                    cnsm[vv] = jnp.max(cnt)
                    return 0

                lax.fori_loop(0, nvh, p1, 0)

                def p2(vv, cur):
                    v = vbase + vv
                    plsc.store_compressed(dst.at[pl.ds(cur, 16)], valfn(v),
                                          mask=maskfn(v))
                    return cur + cnsm[vv]

                return lax.fori_loop(0, nvh, p2, cur)

            return lax.fori_loop(0, nsub, sub, cursor)

        nh = jnp.int32(0)
        for c in range(_NIDX // 2048):
            src_hbm = widx_hbm if c < _BATCH // 2048 else cidx_hbm
            pltpu.sync_copy(
                src_hbm.at[pl.ds((c * 2048) % _BATCH, 2048)], idxc)

            def maskfn(v):
                r = idxc[pl.ds(v * 16, 16)]
                return (r >= lo) & (r < hi)

            def valfn(v, c=c):
                r = idxc[pl.ds(v * 16, 16)]
                pos = (c * 2048 + v * 16) + iota16
                return (pos << 15) | (r - lo)

            nh = compact(128, maskfn, valfn, hits, nh)

        nv = (nh + 15) >> 4
        cur = jnp.int32(0)
        for b in range(16):
            sm[b] = cur

            def maskfn(v, b=b):
                h = hits[pl.ds(v * 16, 16)]
                valid = (v * 16 + iota16) < nh
                return valid & (((h & 0x7FFF) >> 11) == b)

            def valfn(v):
                return hits[pl.ds(v * 16, 16)]

            cur = compact(nv, maskfn, valfn, bkt, cur)
        sm[16] = cur

        match = hits

        def flush():
            pltpu.async_copy(
                big, rows_hbm.at[plsc.Indices(posr, ignored_value=-1)],
                semf).wait()
            neg = jnp.full((16,), -1, jnp.int32)
            for q in range(_SCAP // 16):
                posr[pl.ds(q * 16, 16)] = neg

        def process_block(j, jok, src, cursor):
            jrel = j - jlo
            b = jrel >> 4
            s = sm[b]
            t = sm[b + 1]
            v0 = s >> 4
            nv = ((t + 15) >> 4) - v0

            def mmask(vv):
                v = v0 + vv
                h = bkt[pl.ds(v * 16, 16)]
                k = v * 16 + iota16
                return jok & (k >= s) & (k < t) & (
                    ((h & 0x7FFF) >> 7) == jrel)

            def mval(vv):
                return bkt[pl.ds((v0 + vv) * 16, 16)]

            nm = compact(nv, mmask, mval, match, jnp.int32(0))

            def ebody(g, cur):
                cur = lax.cond(cur > _FLUSH_AT,
                               lambda: (flush(), jnp.int32(0))[1],
                               lambda: cur)
                h = match[pl.ds(g * 16, 16)]
                valid = (g * 16 + iota16) < nm
                posr[pl.ds(cur, 16)] = jnp.where(valid, h >> 15, -1)
                nmg = jnp.minimum(16, nm - g * 16)

                def hbody(i, _, g=g):
                    hsp = plsc.load_gather(
                        match, [jnp.full((16,), 0, jnp.int32) + (g * 16 + i)])
                    lane = hsp & 127
                    for k in range(4):
                        vals = plsc.load_gather(src, [evs[k], lane])
                        big[cur + i, pl.ds(k * 16, 16)] = vals
                    return 0

                lax.fori_loop(0, nmg, hbody, 0)
                return cur + 16

            return lax.fori_loop(0, (nm + 15) >> 4, ebody, cursor)

        neg = jnp.full((16,), -1, jnp.int32)
        for q in range(_SCAP // 16):
            posr[pl.ds(q * 16, 16)] = neg

        def quad(ii, cursor):
            j0 = jlo + _NSLOT * ii
            for s4 in range(_NSLOT):
                pltpu.make_async_copy(
                    tablet_hbm.at[:, pl.ds(pl.multiple_of(0, _BS), _BS)],
                    bufs[s4], sems[s4]).wait()
                cursor = process_block(j0 + s4, (j0 + s4) < _NBF,
                                       bufs[s4], cursor)
                fire(s4, j0 + s4 + _NSLOT)
            return cursor

        cursor = lax.fori_loop(0, _QUADS, quad, jnp.int32(0))

        cursor = lax.cond(wid == _NW - 1,
                          lambda c: process_block(jnp.int32(_NBF), True,
                                                  tail_v, c),
                          lambda c: c, cursor)

        flush()

        dummy = tablet_hbm.at[:, pl.ds(pl.multiple_of(0, _BS), _BS)]
        for s4 in range(_NSLOT):
            pltpu.make_async_copy(dummy, bufs[s4], sems[s4]).wait()

    return phase_a


def _phase_b_body(wref, cref, wscal, bscal, oref):
    s = jnp.sum((wref[...] * cref[...])[:, :_EMBED], axis=1, keepdims=True)
    z = s * wscal[0, 0] + bscal[0, 0]
    oref[...] = 1.0 / (1.0 + jnp.exp(-z))


def _make_phase_b():
    blk = 4096
    grid = _BATCH // blk
    return pl.pallas_call(
        _phase_b_body,
        grid=(grid,),
        in_specs=[
            pl.BlockSpec((blk, 128), lambda i: (i, 0)),
            pl.BlockSpec((blk, 128), lambda i: (i + grid, 0)),
            pl.BlockSpec((1, 1), lambda i: (0, 0), memory_space=pltpu.SMEM),
            pl.BlockSpec((1, 1), lambda i: (0, 0), memory_space=pltpu.SMEM),
        ],
        out_specs=pl.BlockSpec((blk, 1), lambda i: (i, 0)),
        out_shape=jax.ShapeDtypeStruct((_BATCH, 1), jnp.float32),
        compiler_params=pltpu.CompilerParams(
            dimension_semantics=("arbitrary",)),
    )


_phase_a = _make_phase_a()
_phase_b = _make_phase_b()


@jax.jit
def kernel(word, context, table, dense_w, dense_b):
    widx = word.reshape(_BATCH).astype(jnp.int32)
    cidx = context.reshape(_BATCH).astype(jnp.int32)
    tablet = table.T
    tail = jnp.pad(table[_TAIL0:].T.astype(jnp.float32), ((0, 0), (0, 64)))
    rows = _phase_a(widx, cidx, tablet, tail)
    out = _phase_b(rows, rows,
                   dense_w.reshape(1, 1).astype(jnp.float32),
                   dense_b.reshape(1, 1).astype(jnp.float32))
    return out


# speedup vs baseline: 3.0537x; 1.0032x over previous
import functools

import jax
import jax.numpy as jnp
from jax import lax
from jax.experimental import pallas as pl
from jax.experimental.pallas import tpu as pltpu
from jax.experimental.pallas import tpu_sc as plsc

_VOCAB = 1000000
_EMBED = 64
_BATCH = 16384
_NW = 32
_BS = 128
_NBF = 7812
_TAIL0 = _NBF * _BS
_NBW = 245
_NSLOT = 5
_QUADS = (_NBW + _NSLOT - 1) // _NSLOT
_NIDX = 2 * _BATCH
_HCAP = _NIDX + 16
_SCAP = 64
_FLUSH_AT = _SCAP - 16


def _make_phase_a():
    mesh = plsc.VectorSubcoreMesh(core_axis_name="c", subcore_axis_name="s")

    @functools.partial(
        pl.kernel,
        mesh=mesh,
        compiler_params=pltpu.CompilerParams(
            needs_layout_passes=False, use_tc_tiling_on_sc=True),
        out_type=jax.ShapeDtypeStruct((_NIDX, 128), jnp.float32),
        scratch_types=[
            pltpu.VMEM((2048,), jnp.int32),
            pltpu.VMEM((_HCAP,), jnp.int32),
            pltpu.VMEM((_HCAP,), jnp.int32),
            pltpu.VMEM((_EMBED, _BS), jnp.float32),
            pltpu.VMEM((_EMBED, _BS), jnp.float32),
            pltpu.VMEM((_EMBED, _BS), jnp.float32),
            pltpu.VMEM((_EMBED, _BS), jnp.float32),
            pltpu.VMEM((_EMBED, _BS), jnp.float32),
            pltpu.VMEM((_EMBED, 128), jnp.float32),
            pltpu.VMEM((_SCAP, 128), jnp.float32),
            pltpu.VMEM((_SCAP,), jnp.int32),
            pltpu.SMEM((17,), jnp.int32),
            pltpu.SMEM((1024,), jnp.int32),
            pltpu.SemaphoreType.DMA,
            pltpu.SemaphoreType.DMA,
            pltpu.SemaphoreType.DMA,
            pltpu.SemaphoreType.DMA,
            pltpu.SemaphoreType.DMA,
            pltpu.SemaphoreType.DMA,
        ],
    )
    def phase_a(widx_hbm, cidx_hbm, tablet_hbm, tail_hbm, rows_hbm,
                idxc, hits, bkt, tb0, tb1, tb2, tb3, tb4, tail_v,
                big, posr, sm, cnsm,
                semd0, semd1, semd2, semd3, semd4, semf):
        wid = lax.axis_index("s") * 2 + lax.axis_index("c")
        jlo = wid * _NBW
        lo = jlo * _BS
        hi = jnp.minimum(lo + _NBW * _BS, _VOCAB)
        iota16 = lax.iota(jnp.int32, 16)
        evs = [iota16 + 16 * k for k in range(4)]

        pltpu.sync_copy(tail_hbm, tail_v)

        bufs = [tb0, tb1, tb2, tb3, tb4]
        sems = [semd0, semd1, semd2, semd3, semd4]

        def fire(slot, j):
            jc = jnp.minimum(j, _NBF - 1)
            off = pl.multiple_of(jc * _BS, _BS)
            return pltpu.async_copy(tablet_hbm.at[:, pl.ds(off, _BS)],
                                    bufs[slot], sems[slot])

        for s4 in range(_NSLOT):
            fire(s4, jlo + s4)

        def compact(nv, maskfn, valfn, dst, cursor):
            nsub = (nv + 1023) >> 10

            def sub(si, cur):
                vbase = si * 1024
                nvh = jnp.minimum(1024, nv - vbase)

                def p1(vv, _):
                    cnt = plsc.all_reduce_population_count(
                        maskfn(vbase + vv))
                    cnsm[vv] = jnp.max(cnt)
                    return 0

                lax.fori_loop(0, nvh, p1, 0)

                def p2(vv, cur):
                    v = vbase + vv
                    plsc.store_compressed(dst.at[pl.ds(cur, 16)], valfn(v),
                                          mask=maskfn(v))
                    return cur + cnsm[vv]

                return lax.fori_loop(0, nvh, p2, cur)

            return lax.fori_loop(0, nsub, sub, cursor)

        nh = jnp.int32(0)
        for c in range(_NIDX // 2048):
            src_hbm = widx_hbm if c < _BATCH // 2048 else cidx_hbm
            pltpu.sync_copy(
                src_hbm.at[pl.ds((c * 2048) % _BATCH, 2048)], idxc)

            def maskfn(v):
                r = idxc[pl.ds(v * 16, 16)]
                return (r >= lo) & (r < hi)

            def valfn(v, c=c):
                r = idxc[pl.ds(v * 16, 16)]
                pos = (c * 2048 + v * 16) + iota16
                return (pos << 15) | (r - lo)

            nh = compact(128, maskfn, valfn, hits, nh)

        nv = (nh + 15) >> 4
        cur = jnp.int32(0)
        for b in range(16):
            sm[b] = cur

            def maskfn(v, b=b):
                h = hits[pl.ds(v * 16, 16)]
                valid = (v * 16 + iota16) < nh
                return valid & (((h & 0x7FFF) >> 11) == b)

            def valfn(v):
                return hits[pl.ds(v * 16, 16)]

            cur = compact(nv, maskfn, valfn, bkt, cur)
        sm[16] = cur

        match = hits

        def flush():
            pltpu.async_copy(
                big, rows_hbm.at[plsc.Indices(posr, ignored_value=-1)],
                semf).wait()
            neg = jnp.full((16,), -1, jnp.int32)
            for q in range(_SCAP // 16):
                posr[pl.ds(q * 16, 16)] = neg

        def process_block(j, jok, src, cursor):
            jrel = j - jlo
            b = jrel >> 4
            s = sm[b]
            t = sm[b + 1]
            v0 = s >> 4
            nv = ((t + 15) >> 4) - v0

            def mmask(vv):
                v = v0 + vv
                h = bkt[pl.ds(v * 16, 16)]
                k = v * 16 + iota16
                return jok & (k >= s) & (k < t) & (
                    ((h & 0x7FFF) >> 7) == jrel)

            def mval(vv):
                return bkt[pl.ds((v0 + vv) * 16, 16)]

            nm = compact(nv, mmask, mval, match, jnp.int32(0))

            def ebody(g, cur):
                cur = lax.cond(cur > _FLUSH_AT,
                               lambda: (flush(), jnp.int32(0))[1],
                               lambda: cur)
                h = match[pl.ds(g * 16, 16)]
                valid = (g * 16 + iota16) < nm
                posr[pl.ds(cur, 16)] = jnp.where(valid, h >> 15, -1)
                nmg = jnp.minimum(16, nm - g * 16)

                def hbody(i, _, g=g):
                    hsp = plsc.load_gather(
                        match, [jnp.full((16,), 0, jnp.int32) + (g * 16 + i)])
                    lane = hsp & 127
                    for k in range(4):
                        vals = plsc.load_gather(src, [evs[k], lane])
                        big[cur + i, pl.ds(k * 16, 16)] = vals
                    return 0

                lax.fori_loop(0, nmg, hbody, 0)
                return cur + 16

            return lax.fori_loop(0, (nm + 15) >> 4, ebody, cursor)

        neg = jnp.full((16,), -1, jnp.int32)
        for q in range(_SCAP // 16):
            posr[pl.ds(q * 16, 16)] = neg

        def quad(ii, cursor):
            j0 = jlo + _NSLOT * ii
            for s4 in range(_NSLOT):
                pltpu.make_async_copy(
                    tablet_hbm.at[:, pl.ds(pl.multiple_of(0, _BS), _BS)],
                    bufs[s4], sems[s4]).wait()
                cursor = process_block(j0 + s4, (j0 + s4) < _NBF,
                                       bufs[s4], cursor)
                fire(s4, j0 + s4 + _NSLOT)
            return cursor

        cursor = lax.fori_loop(0, _QUADS, quad, jnp.int32(0))

        cursor = lax.cond(wid == _NW - 1,
                          lambda c: process_block(jnp.int32(_NBF), True,
                                                  tail_v, c),
                          lambda c: c, cursor)

        flush()

        dummy = tablet_hbm.at[:, pl.ds(pl.multiple_of(0, _BS), _BS)]
        for s4 in range(_NSLOT):
            pltpu.make_async_copy(dummy, bufs[s4], sems[s4]).wait()

    return phase_a


def _phase_b_body(wref, cref, wscal, bscal, oref):
    s = jnp.sum((wref[...] * cref[...])[:, :_EMBED], axis=1, keepdims=True)
    z = s * wscal[0, 0] + bscal[0, 0]
    oref[...] = 1.0 / (1.0 + jnp.exp(-z))


def _make_phase_b():
    blk = 8192
    grid = _BATCH // blk
    return pl.pallas_call(
        _phase_b_body,
        grid=(grid,),
        in_specs=[
            pl.BlockSpec((blk, 128), lambda i: (i, 0)),
            pl.BlockSpec((blk, 128), lambda i: (i + grid, 0)),
            pl.BlockSpec((1, 1), lambda i: (0, 0), memory_space=pltpu.SMEM),
            pl.BlockSpec((1, 1), lambda i: (0, 0), memory_space=pltpu.SMEM),
        ],
        out_specs=pl.BlockSpec((blk, 1), lambda i: (i, 0)),
        out_shape=jax.ShapeDtypeStruct((_BATCH, 1), jnp.float32),
        compiler_params=pltpu.CompilerParams(
            dimension_semantics=("arbitrary",)),
    )


_phase_a = _make_phase_a()
_phase_b = _make_phase_b()


@jax.jit
def kernel(word, context, table, dense_w, dense_b):
    widx = word.reshape(_BATCH).astype(jnp.int32)
    cidx = context.reshape(_BATCH).astype(jnp.int32)
    tablet = table.T
    tail = jnp.pad(table[_TAIL0:].T.astype(jnp.float32), ((0, 0), (0, 64)))
    rows = _phase_a(widx, cidx, tablet, tail)
    out = _phase_b(rows, rows,
                   dense_w.reshape(1, 1).astype(jnp.float32),
                   dense_b.reshape(1, 1).astype(jnp.float32))
    return out
